# baseline (speedup 1.0000x reference)
import functools

import jax
import jax.numpy as jnp
import numpy as np
from jax import lax
from jax.experimental import pallas as pl
from jax.experimental.pallas import tpu as pltpu
from jax.experimental.pallas import tpu_sc as plsc

F32 = jnp.float32
I32 = jnp.int32

E = 8
ENC = 256
WID = 256
NXF = 10
NDF = 4
T = 256

SC_CORES = 2
SC_SUBCORES = 16
NWORK = SC_CORES * SC_SUBCORES
CHUNK = 128


BF16 = jnp.bfloat16


def _split3(a):
    ah = a.astype(BF16)
    al = (a - ah.astype(F32)).astype(BF16)
    return ah, al


def _dot3(a, b):
    ah, al = _split3(a)
    bh, bl = _split3(b)
    return (jnp.dot(ah, bh, preferred_element_type=F32)
            + jnp.dot(ah, bl, preferred_element_type=F32)
            + jnp.dot(al, bh, preferred_element_type=F32))


def _dot3_pre(ah, al, bh, bl):
    return (jnp.dot(ah, bh, preferred_element_type=F32)
            + jnp.dot(ah, bl, preferred_element_type=F32)
            + jnp.dot(al, bh, preferred_element_type=F32))


def _pe_matrix(degree, width):
    m = np.zeros((3, width), np.float32)
    half = width // 2
    for d in range(degree):
        for c in range(3):
            m[c, 3 * d + c] = 2.0 ** d
            m[c, half + 3 * d + c] = 2.0 ** d
    return jnp.asarray(m)


def _pe_sincos(x, mat):
    t = (x[:, 0:1] * mat[0:1, :] + x[:, 1:2] * mat[1:2, :]
         + x[:, 2:3] * mat[2:3, :])
    half = t.shape[1] // 2
    return jnp.sin(t[:, :half]), jnp.cos(t[:, half:])


def _gating_body(temp_ref, xyz_ref, pemat_ref, wenc_ref, wid_ref, benc_ref,
                 wg_ref, bg_ref,
                 ya_ref, yb_ref, gates_ref, onehot_ref, gtop_ref,
                 counts_ref, gsum_ref):
    i = pl.program_id(0)
    x = xyz_ref[...]
    s, c = _pe_sincos(x, pemat_ref[...])
    y = (jnp.dot(s, wenc_ref[0], preferred_element_type=F32)
         + jnp.dot(c, wenc_ref[1], preferred_element_type=F32)
         + jnp.dot(x, wid_ref[...], preferred_element_type=F32)
         + benc_ref[...])
    ya_ref[...] = y[:, :128]
    yb_ref[...] = y[:, 128:]
    logits = jnp.dot(y, wg_ref[...], preferred_element_type=F32) + bg_ref[...]
    lt = logits / temp_ref[0, 0]
    m = jnp.max(lt, axis=1, keepdims=True)
    ex = jnp.exp(lt - m)
    g = ex / jnp.sum(ex, axis=1, keepdims=True)
    gates_ref[...] = g
    li = lax.broadcasted_iota(I32, g.shape, 1)
    gm = jnp.max(g, axis=1, keepdims=True)
    am = jnp.min(jnp.where(g == gm, li, E), axis=1, keepdims=True)
    oh = (li == am).astype(F32)
    onehot_ref[...] = oh
    gtop_ref[...] = gm

    @pl.when(i == 0)
    def _():
        counts_ref[...] = jnp.zeros_like(counts_ref)
        gsum_ref[...] = jnp.zeros_like(gsum_ref)

    counts_ref[...] += jnp.sum(oh, axis=0, keepdims=True)
    gsum_ref[...] += jnp.sum(g, axis=0, keepdims=True)


def _gating(xyz2, temp11, wenc_sc, wid3, b_enc, W_g, b_g, n):
    ba = 1024
    grid = (n // ba,)
    return pl.pallas_call(
        _gating_body,
        grid=grid,
        in_specs=[
            pl.BlockSpec(memory_space=pltpu.SMEM),
            pl.BlockSpec((ba, 3), lambda i: (i, 0)),
            pl.BlockSpec((3, 64), lambda i: (0, 0)),
            pl.BlockSpec((2, 32, ENC), lambda i: (0, 0, 0)),
            pl.BlockSpec((3, ENC), lambda i: (0, 0)),
            pl.BlockSpec((1, ENC), lambda i: (0, 0)),
            pl.BlockSpec((ENC, E), lambda i: (0, 0)),
            pl.BlockSpec((1, E), lambda i: (0, 0)),
        ],
        out_specs=[
            pl.BlockSpec((ba, 128), lambda i: (i, 0)),
            pl.BlockSpec((ba, 128), lambda i: (i, 0)),
            pl.BlockSpec((ba, E), lambda i: (i, 0)),
            pl.BlockSpec((ba, E), lambda i: (i, 0)),
            pl.BlockSpec((ba, 1), lambda i: (i, 0)),
            pl.BlockSpec((1, E), lambda i: (0, 0)),
            pl.BlockSpec((1, E), lambda i: (0, 0)),
        ],
        out_shape=[
            jax.ShapeDtypeStruct((n, 128), F32),
            jax.ShapeDtypeStruct((n, 128), F32),
            jax.ShapeDtypeStruct((n, E), F32),
            jax.ShapeDtypeStruct((n, E), F32),
            jax.ShapeDtypeStruct((n, 1), F32),
            jax.ShapeDtypeStruct((1, E), F32),
            jax.ShapeDtypeStruct((1, E), F32),
        ],
    )(temp11, xyz2, _pe_matrix(NXF, 64), wenc_sc, wid3,
      b_enc.reshape(1, ENC), W_g, b_g.reshape(1, E))


def _dest_body(onehot_ref, starts_ref, ltri_ref, dest_ref, carry_ref):
    i = pl.program_id(0)

    @pl.when(i == 0)
    def _():
        carry_ref[...] = jnp.zeros_like(carry_ref)

    oh = onehot_ref[...]
    ranks = jnp.dot(ltri_ref[...], oh, preferred_element_type=F32,
                    precision=lax.Precision.DEFAULT)
    base = starts_ref[...] + carry_ref[...]
    destf = jnp.sum(oh * (base + ranks), axis=1, keepdims=True)
    dest_ref[...] = destf.astype(I32)
    carry_ref[...] += jnp.sum(oh, axis=0, keepdims=True)


def _dest(onehot, starts18, n):
    tb = 512
    r = np.arange(tb)
    ltri = jnp.asarray((r[:, None] > r[None, :]).astype(np.float32))
    return pl.pallas_call(
        _dest_body,
        grid=(n // tb,),
        in_specs=[
            pl.BlockSpec((tb, E), lambda i: (i, 0)),
            pl.BlockSpec((1, E), lambda i: (0, 0)),
            pl.BlockSpec((tb, tb), lambda i: (0, 0)),
        ],
        out_specs=pl.BlockSpec((tb, 1), lambda i: (i, 0)),
        out_shape=jax.ShapeDtypeStruct((n, 1), I32),
        scratch_shapes=[pltpu.VMEM((1, E), F32)],
    )(onehot, starts18, ltri)


def _dispatch_scatter(ya, yb, dest3, npad):
    n = ya.shape[0]
    per_w = n // NWORK
    nchunks = per_w // CHUNK
    mesh = plsc.VectorSubcoreMesh(core_axis_name="c", subcore_axis_name="s")

    @functools.partial(
        pl.kernel,
        mesh=mesh,
        out_type=[jax.ShapeDtypeStruct((npad, 128), F32),
                  jax.ShapeDtypeStruct((npad, 128), F32)],
        scratch_types=[
            pltpu.VMEM((nchunks, CHUNK), I32),
            pltpu.VMEM((CHUNK, 128), F32),
            pltpu.VMEM((CHUNK, 128), F32),
            pltpu.SemaphoreType.DMA,
            pltpu.SemaphoreType.DMA,
        ],
    )
    def k(ya_hbm, yb_hbm, dest_hbm, ysa_hbm, ysb_hbm, idx_v, rowa_v, rowb_v,
          sema, semb):
        wid = lax.axis_index("s") * SC_CORES + lax.axis_index("c")
        pltpu.sync_copy(dest_hbm.at[wid], idx_v)
        base = wid * per_w
        for j in range(nchunks):
            pltpu.sync_copy(ya_hbm.at[pl.ds(base + j * CHUNK, CHUNK)], rowa_v)
            pltpu.sync_copy(yb_hbm.at[pl.ds(base + j * CHUNK, CHUNK)], rowb_v)
            ca = pltpu.async_copy(rowa_v, ysa_hbm.at[idx_v.at[j]], sema)
            cb = pltpu.async_copy(rowb_v, ysb_hbm.at[idx_v.at[j]], semb)
            ca.wait()
            cb.wait()

    return k(ya, yb, dest3)


def _expert_body(eid_ref, ysa_ref, ysb_ref, w1_ref, b1_ref, w2_ref, b2_ref,
                 w3_ref, b3_ref, w4_ref, b4_ref, outa_ref, outb_ref):
    t = pl.program_id(0)
    e = eid_ref[t]
    a = jnp.concatenate([ysa_ref[...], ysb_ref[...]], axis=1)
    h = jnp.maximum(jnp.dot(a, w1_ref[e], preferred_element_type=F32) + b1_ref[e], 0.0)
    h = jnp.maximum(jnp.dot(h, w2_ref[e], preferred_element_type=F32) + b2_ref[e], 0.0)
    h = jnp.maximum(jnp.dot(h, w3_ref[e], preferred_element_type=F32) + b3_ref[e], 0.0)
    h = jnp.dot(h, w4_ref[e], preferred_element_type=F32) + b4_ref[e]
    outa_ref[...] = h[:, :128]
    outb_ref[...] = h[:, 128:]


def _experts(tile_eid, ysa, ysb, We1, be1, We2, be2, We3, be3, We4, be4,
             npad):
    nt = npad // T
    wspec = pl.BlockSpec((E, ENC, WID), lambda t, eid: (0, 0, 0))
    bspec = pl.BlockSpec((E, 1, WID), lambda t, eid: (0, 0, 0))
    grid_spec = pltpu.PrefetchScalarGridSpec(
        num_scalar_prefetch=1,
        grid=(nt,),
        in_specs=[
            pl.BlockSpec((T, 128), lambda t, eid: (t, 0)),
            pl.BlockSpec((T, 128), lambda t, eid: (t, 0)),
            wspec, bspec, wspec, bspec, wspec, bspec, wspec, bspec,
        ],
        out_specs=[pl.BlockSpec((T, 128), lambda t, eid: (t, 0)),
                   pl.BlockSpec((T, 128), lambda t, eid: (t, 0))],
    )
    return pl.pallas_call(
        _expert_body,
        grid_spec=grid_spec,
        out_shape=[jax.ShapeDtypeStruct((npad, 128), F32),
                   jax.ShapeDtypeStruct((npad, 128), F32)],
    )(tile_eid, ysa, ysb,
      We1, be1.reshape(E, 1, WID), We2, be2.reshape(E, 1, WID),
      We3, be3.reshape(E, 1, WID), We4, be4.reshape(E, 1, WID))


def _combine_gather(hsa, hsb, dest3, n):
    per_w = n // NWORK
    nchunks = per_w // CHUNK
    mesh = plsc.VectorSubcoreMesh(core_axis_name="c", subcore_axis_name="s")

    @functools.partial(
        pl.kernel,
        mesh=mesh,
        out_type=[jax.ShapeDtypeStruct((n, 128), F32),
                  jax.ShapeDtypeStruct((n, 128), F32)],
        scratch_types=[
            pltpu.VMEM((nchunks, CHUNK), I32),
            pltpu.VMEM((CHUNK, 128), F32),
            pltpu.VMEM((CHUNK, 128), F32),
            pltpu.SemaphoreType.DMA,
            pltpu.SemaphoreType.DMA,
        ],
    )
    def k(hsa_hbm, hsb_hbm, dest_hbm, outa_hbm, outb_hbm, idx_v, rowa_v,
          rowb_v, sema, semb):
        wid = lax.axis_index("s") * SC_CORES + lax.axis_index("c")
        pltpu.sync_copy(dest_hbm.at[wid], idx_v)
        base = wid * per_w
        for j in range(nchunks):
            ca = pltpu.async_copy(hsa_hbm.at[idx_v.at[j]], rowa_v, sema)
            cb = pltpu.async_copy(hsb_hbm.at[idx_v.at[j]], rowb_v, semb)
            ca.wait()
            cb.wait()
            pltpu.sync_copy(rowa_v, outa_hbm.at[pl.ds(base + j * CHUNK, CHUNK)])
            pltpu.sync_copy(rowb_v, outb_hbm.at[pl.ds(base + j * CHUNK, CHUNK)])

    return k(hsa, hsb, dest3)


def _head_body(hrawa_ref, hrawb_ref, gtop_ref, vdir_ref, pemat_ref,
               wr1az_ref, wvs_ref, wvc_ref, wvi_ref, br1z_ref,
               wr2_ref, br2_ref,
               sig_ref, rgb_ref, ssum_ref):
    i = pl.program_id(0)
    so = jnp.concatenate([hrawa_ref[...], hrawb_ref[...]],
                         axis=1) * gtop_ref[...]
    v = vdir_ref[...]
    s, c = _pe_sincos(v, pemat_ref[...])
    u = (jnp.dot(so, wr1az_ref[...], preferred_element_type=F32)
         + jnp.dot(s, wvs_ref[...], preferred_element_type=F32)
         + jnp.dot(c, wvc_ref[...], preferred_element_type=F32)
         + jnp.dot(v, wvi_ref[...], preferred_element_type=F32)
         + br1z_ref[...])
    z = u[:, 128:129]
    sig = jnp.maximum(z, 0.0) + jnp.log(1.0 + jnp.exp(-jnp.abs(z)))
    sig_ref[...] = sig
    hr = jnp.maximum(u[:, :128], 0.0)
    t = jnp.dot(hr, wr2_ref[...], preferred_element_type=F32) + br2_ref[...]
    rgb_ref[...] = 1.0 / (1.0 + jnp.exp(-t))

    @pl.when(i == 0)
    def _():
        ssum_ref[...] = jnp.zeros_like(ssum_ref)

    ssum_ref[...] += jnp.sum(sig, axis=0, keepdims=True)


def _heads(hrawa, hrawb, gtop, vdir2, wr1az, wvs, wvc, wvi, br1z, wr2p,
           br2p, n):
    bf = 1024
    return pl.pallas_call(
        _head_body,
        grid=(n // bf,),
        in_specs=[
            pl.BlockSpec((bf, 128), lambda i: (i, 0)),
            pl.BlockSpec((bf, 128), lambda i: (i, 0)),
            pl.BlockSpec((bf, 1), lambda i: (i, 0)),
            pl.BlockSpec((bf, 3), lambda i: (i, 0)),
            pl.BlockSpec((3, 32), lambda i: (0, 0)),
            pl.BlockSpec((ENC, 256), lambda i: (0, 0)),
            pl.BlockSpec((16, 256), lambda i: (0, 0)),
            pl.BlockSpec((16, 256), lambda i: (0, 0)),
            pl.BlockSpec((3, 256), lambda i: (0, 0)),
            pl.BlockSpec((1, 256), lambda i: (0, 0)),
            pl.BlockSpec((128, 128), lambda i: (0, 0)),
            pl.BlockSpec((1, 128), lambda i: (0, 0)),
        ],
        out_specs=[
            pl.BlockSpec((bf, 1), lambda i: (i, 0)),
            pl.BlockSpec((bf, 128), lambda i: (i, 0)),
            pl.BlockSpec((1, 1), lambda i: (0, 0)),
        ],
        out_shape=[
            jax.ShapeDtypeStruct((n, 1), F32),
            jax.ShapeDtypeStruct((n, 128), F32),
            jax.ShapeDtypeStruct((1, 1), F32),
        ],
    )(hrawa, hrawb, gtop, vdir2, _pe_matrix(NDF, 32), wr1az, wvs, wvc, wvi,
      br1z, wr2p, br2p)


def kernel(xyz, viewdir, shape_latent, texture_latent, temperature,
           W_enc, b_enc, W_g, b_g,
           We1, be1, We2, be2, We3, be3, We4, be4,
           W_sig, b_sig, W_r1, b_r1, W_r2, b_r2):
    nrays, nsamples, _ = xyz.shape
    n = nrays * nsamples
    npad = (n // T + E) * T

    xyz2 = xyz.reshape(n, 3)
    vdir2 = viewdir.reshape(n, 3)
    temp11 = temperature.reshape(1, 1)
    nsf = 3 * NXF
    z2 = jnp.zeros((32 - nsf, ENC), F32)
    wenc_sc = jnp.stack([
        jnp.concatenate([W_enc[3:3 + nsf], z2], axis=0),
        jnp.concatenate([W_enc[3 + nsf:3 + 2 * nsf], z2], axis=0),
    ])
    wid3 = W_enc[:3]

    ya, yb, gates, onehot, gtop, counts, gsum = _gating(
        xyz2, temp11, wenc_sc, wid3, b_enc, W_g, b_g, n)

    cnt = counts.reshape(E)
    tile_cnt = jnp.ceil(cnt / T).astype(I32)
    tile_start = jnp.concatenate(
        [jnp.zeros((1,), I32), jnp.cumsum(tile_cnt)[:-1]])
    starts18 = (tile_start * T).astype(F32).reshape(1, E)
    nt = npad // T
    cum = jnp.cumsum(tile_cnt)
    tidx = jnp.arange(nt, dtype=I32)
    tile_eid = jnp.minimum(
        jnp.sum((tidx[:, None] >= cum[None, :]).astype(I32), axis=1),
        E - 1).astype(I32)

    dest = _dest(onehot, starts18, n)
    dest3 = dest.reshape(NWORK, (n // NWORK) // CHUNK, CHUNK)

    ysa, ysb = _dispatch_scatter(ya, yb, dest3, npad)
    hsa, hsb = _experts(tile_eid, ysa, ysb, We1, be1, We2, be2, We3, be3,
                        We4, be4, npad)
    hrawa, hrawb = _combine_gather(hsa, hsb, dest3, n)

    ncf = 3 * NDF
    wr1az = jnp.concatenate(
        [W_r1[:ENC], W_sig, jnp.zeros((ENC, 127), F32)], axis=1)
    wvs = jnp.zeros((16, 256), F32).at[:ncf, :128].set(W_r1[ENC + 3:ENC + 3 + ncf])
    wvc = jnp.zeros((16, 256), F32).at[:ncf, :128].set(W_r1[ENC + 3 + ncf:])
    wvi = jnp.zeros((3, 256), F32).at[:, :128].set(W_r1[ENC:ENC + 3])
    br1z = jnp.concatenate(
        [b_r1, b_sig, jnp.zeros((127,), F32)]).reshape(1, 256)
    wr2p = jnp.concatenate([W_r2, jnp.zeros((128, 125), F32)], axis=1)
    br2p = jnp.concatenate([b_r2, jnp.zeros((125,), F32)]).reshape(1, 128)

    sig, rgbp, ssum = _heads(hrawa, hrawb, gtop, vdir2, wr1az, wvs, wvc,
                             wvi, br1z, wr2p, br2p, n)

    sigmas = sig.reshape(nrays, nsamples, 1)
    rgbs = rgbp[:, :3].reshape(nrays, nsamples, 3)
    gates_soft_o = gates.reshape(nrays, nsamples, E)
    gates_hard_o = onehot.reshape(nrays, nsamples, E)
    mean_sigma = (ssum / n).reshape(1)
    num_pts = cnt
    aux_loss = E * jnp.sum((cnt / n) * (gsum.reshape(E) / n))
    return (sigmas, rgbs, gates_soft_o, gates_hard_o,
            mean_sigma, num_pts, aux_loss)


# speedup vs baseline: 1.2850x; 1.2850x over previous
import functools

import jax
import jax.numpy as jnp
import numpy as np
from jax import lax
from jax.experimental import pallas as pl
from jax.experimental.pallas import tpu as pltpu
from jax.experimental.pallas import tpu_sc as plsc

F32 = jnp.float32
I32 = jnp.int32

E = 8
ENC = 256
WID = 256
NXF = 10
NDF = 4
T = 256

SC_CORES = 2
SC_SUBCORES = 16
NWORK = SC_CORES * SC_SUBCORES
CHUNK = 128


BF16 = jnp.bfloat16


def _split3(a):
    ah = a.astype(BF16)
    al = (a - ah.astype(F32)).astype(BF16)
    return ah, al


def _dot3(a, b):
    ah, al = _split3(a)
    bh, bl = _split3(b)
    return (jnp.dot(ah, bh, preferred_element_type=F32)
            + jnp.dot(ah, bl, preferred_element_type=F32)
            + jnp.dot(al, bh, preferred_element_type=F32))


def _dot3_pre(ah, al, bh, bl):
    return (jnp.dot(ah, bh, preferred_element_type=F32)
            + jnp.dot(ah, bl, preferred_element_type=F32)
            + jnp.dot(al, bh, preferred_element_type=F32))


def _pe_matrix(degree, width):
    m = np.zeros((3, width), np.float32)
    half = width // 2
    for d in range(degree):
        for c in range(3):
            m[c, 3 * d + c] = 2.0 ** d
            m[c, half + 3 * d + c] = 2.0 ** d
    return jnp.asarray(m)


def _pe_sincos(x, mat):
    t = (x[:, 0:1] * mat[0:1, :] + x[:, 1:2] * mat[1:2, :]
         + x[:, 2:3] * mat[2:3, :])
    half = t.shape[1] // 2
    return jnp.sin(t[:, :half]), jnp.cos(t[:, half:])


_DN0 = (((0,), (0,)), ((), ()))


def _gating_body(temp_ref, xyzT_ref, wsin_ref, wcos_ref, wid_ref, benc_ref,
                 wg_ref, bg_ref,
                 ya_ref, yb_ref, gates_ref, onehot_ref, gtop_ref,
                 counts_ref, gsum_ref):
    i = pl.program_id(0)
    xt = xyzT_ref[...]
    t30 = jnp.concatenate([xt * (2.0 ** d) for d in range(NXF)], axis=0)
    s = jnp.sin(t30)
    c = jnp.cos(t30)
    y = (lax.dot_general(s, wsin_ref[...], _DN0, preferred_element_type=F32)
         + lax.dot_general(c, wcos_ref[...], _DN0, preferred_element_type=F32)
         + lax.dot_general(xt, wid_ref[...], _DN0, preferred_element_type=F32)
         + benc_ref[...])
    ya_ref[...] = y[:, :128]
    yb_ref[...] = y[:, 128:]
    logits = jnp.dot(y, wg_ref[...], preferred_element_type=F32) + bg_ref[...]
    lt = logits / temp_ref[0, 0]
    m = jnp.max(lt, axis=1, keepdims=True)
    ex = jnp.exp(lt - m)
    g = ex / jnp.sum(ex, axis=1, keepdims=True)
    gates_ref[...] = g
    li = lax.broadcasted_iota(I32, g.shape, 1)
    gm = jnp.max(g, axis=1, keepdims=True)
    am = jnp.min(jnp.where(g == gm, li, E), axis=1, keepdims=True)
    oh = (li == am).astype(F32)
    onehot_ref[...] = oh
    gtop_ref[...] = gm

    @pl.when(i == 0)
    def _():
        counts_ref[...] = jnp.zeros_like(counts_ref)
        gsum_ref[...] = jnp.zeros_like(gsum_ref)

    counts_ref[...] += jnp.sum(oh, axis=0, keepdims=True)
    gsum_ref[...] += jnp.sum(g, axis=0, keepdims=True)


def _gating(xyzT, temp11, wsin30, wcos30, wid3, b_enc, W_g, b_g, n):
    ba = 1024
    grid = (n // ba,)
    return pl.pallas_call(
        _gating_body,
        grid=grid,
        in_specs=[
            pl.BlockSpec(memory_space=pltpu.SMEM),
            pl.BlockSpec((3, ba), lambda i: (0, i)),
            pl.BlockSpec((3 * NXF, ENC), lambda i: (0, 0)),
            pl.BlockSpec((3 * NXF, ENC), lambda i: (0, 0)),
            pl.BlockSpec((3, ENC), lambda i: (0, 0)),
            pl.BlockSpec((1, ENC), lambda i: (0, 0)),
            pl.BlockSpec((ENC, E), lambda i: (0, 0)),
            pl.BlockSpec((1, E), lambda i: (0, 0)),
        ],
        out_specs=[
            pl.BlockSpec((ba, 128), lambda i: (i, 0)),
            pl.BlockSpec((ba, 128), lambda i: (i, 0)),
            pl.BlockSpec((ba, E), lambda i: (i, 0)),
            pl.BlockSpec((ba, E), lambda i: (i, 0)),
            pl.BlockSpec((ba, 1), lambda i: (i, 0)),
            pl.BlockSpec((1, E), lambda i: (0, 0)),
            pl.BlockSpec((1, E), lambda i: (0, 0)),
        ],
        out_shape=[
            jax.ShapeDtypeStruct((n, 128), F32),
            jax.ShapeDtypeStruct((n, 128), F32),
            jax.ShapeDtypeStruct((n, E), F32),
            jax.ShapeDtypeStruct((n, E), F32),
            jax.ShapeDtypeStruct((n, 1), F32),
            jax.ShapeDtypeStruct((1, E), F32),
            jax.ShapeDtypeStruct((1, E), F32),
        ],
    )(temp11, xyzT, wsin30, wcos30, wid3,
      b_enc.reshape(1, ENC), W_g, b_g.reshape(1, E))


def _dest_body(onehot_ref, starts_ref, ltri_ref, dest_ref, carry_ref):
    i = pl.program_id(0)

    @pl.when(i == 0)
    def _():
        carry_ref[...] = jnp.zeros_like(carry_ref)

    oh = onehot_ref[...]
    ranks = jnp.dot(ltri_ref[...], oh, preferred_element_type=F32,
                    precision=lax.Precision.DEFAULT)
    base = starts_ref[...] + carry_ref[...]
    destf = jnp.sum(oh * (base + ranks), axis=1, keepdims=True)
    dest_ref[...] = destf.astype(I32)
    carry_ref[...] += jnp.sum(oh, axis=0, keepdims=True)


def _dest(onehot, starts18, n):
    tb = 512
    r = np.arange(tb)
    ltri = jnp.asarray((r[:, None] > r[None, :]).astype(np.float32))
    return pl.pallas_call(
        _dest_body,
        grid=(n // tb,),
        in_specs=[
            pl.BlockSpec((tb, E), lambda i: (i, 0)),
            pl.BlockSpec((1, E), lambda i: (0, 0)),
            pl.BlockSpec((tb, tb), lambda i: (0, 0)),
        ],
        out_specs=pl.BlockSpec((tb, 1), lambda i: (i, 0)),
        out_shape=jax.ShapeDtypeStruct((n, 1), I32),
        scratch_shapes=[pltpu.VMEM((1, E), F32)],
    )(onehot, starts18, ltri)


def _dispatch_scatter(ya, yb, dest3, npad):
    n = ya.shape[0]
    per_w = n // NWORK
    nchunks = per_w // CHUNK
    mesh = plsc.VectorSubcoreMesh(core_axis_name="c", subcore_axis_name="s")

    @functools.partial(
        pl.kernel,
        mesh=mesh,
        out_type=[jax.ShapeDtypeStruct((npad, 128), F32),
                  jax.ShapeDtypeStruct((npad, 128), F32)],
        scratch_types=[
            pltpu.VMEM((nchunks, CHUNK), I32),
            pltpu.VMEM((CHUNK, 128), F32),
            pltpu.VMEM((CHUNK, 128), F32),
            pltpu.SemaphoreType.DMA,
            pltpu.SemaphoreType.DMA,
        ],
    )
    def k(ya_hbm, yb_hbm, dest_hbm, ysa_hbm, ysb_hbm, idx_v, rowa_v, rowb_v,
          sema, semb):
        wid = lax.axis_index("s") * SC_CORES + lax.axis_index("c")
        pltpu.sync_copy(dest_hbm.at[wid], idx_v)
        base = wid * per_w
        for j in range(nchunks):
            pltpu.sync_copy(ya_hbm.at[pl.ds(base + j * CHUNK, CHUNK)], rowa_v)
            pltpu.sync_copy(yb_hbm.at[pl.ds(base + j * CHUNK, CHUNK)], rowb_v)
            ca = pltpu.async_copy(rowa_v, ysa_hbm.at[idx_v.at[j]], sema)
            cb = pltpu.async_copy(rowb_v, ysb_hbm.at[idx_v.at[j]], semb)
            ca.wait()
            cb.wait()

    return k(ya, yb, dest3)


def _expert_body(eid_ref, ysa_ref, ysb_ref, w1_ref, b1_ref, w2_ref, b2_ref,
                 w3_ref, b3_ref, w4_ref, b4_ref, outa_ref, outb_ref):
    t = pl.program_id(0)
    e = eid_ref[t]
    a = jnp.concatenate([ysa_ref[...], ysb_ref[...]], axis=1)
    h = jnp.maximum(jnp.dot(a, w1_ref[e], preferred_element_type=F32) + b1_ref[e], 0.0)
    h = jnp.maximum(jnp.dot(h, w2_ref[e], preferred_element_type=F32) + b2_ref[e], 0.0)
    h = jnp.maximum(jnp.dot(h, w3_ref[e], preferred_element_type=F32) + b3_ref[e], 0.0)
    h = jnp.dot(h, w4_ref[e], preferred_element_type=F32) + b4_ref[e]
    outa_ref[...] = h[:, :128]
    outb_ref[...] = h[:, 128:]


def _experts(tile_eid, ysa, ysb, We1, be1, We2, be2, We3, be3, We4, be4,
             npad):
    nt = npad // T
    wspec = pl.BlockSpec((E, ENC, WID), lambda t, eid: (0, 0, 0))
    bspec = pl.BlockSpec((E, 1, WID), lambda t, eid: (0, 0, 0))
    grid_spec = pltpu.PrefetchScalarGridSpec(
        num_scalar_prefetch=1,
        grid=(nt,),
        in_specs=[
            pl.BlockSpec((T, 128), lambda t, eid: (t, 0)),
            pl.BlockSpec((T, 128), lambda t, eid: (t, 0)),
            wspec, bspec, wspec, bspec, wspec, bspec, wspec, bspec,
        ],
        out_specs=[pl.BlockSpec((T, 128), lambda t, eid: (t, 0)),
                   pl.BlockSpec((T, 128), lambda t, eid: (t, 0))],
    )
    return pl.pallas_call(
        _expert_body,
        grid_spec=grid_spec,
        out_shape=[jax.ShapeDtypeStruct((npad, 128), F32),
                   jax.ShapeDtypeStruct((npad, 128), F32)],
    )(tile_eid, ysa, ysb,
      We1, be1.reshape(E, 1, WID), We2, be2.reshape(E, 1, WID),
      We3, be3.reshape(E, 1, WID), We4, be4.reshape(E, 1, WID))


def _combine_gather(hsa, hsb, dest3, n):
    per_w = n // NWORK
    nchunks = per_w // CHUNK
    mesh = plsc.VectorSubcoreMesh(core_axis_name="c", subcore_axis_name="s")

    @functools.partial(
        pl.kernel,
        mesh=mesh,
        out_type=[jax.ShapeDtypeStruct((n, 128), F32),
                  jax.ShapeDtypeStruct((n, 128), F32)],
        scratch_types=[
            pltpu.VMEM((nchunks, CHUNK), I32),
            pltpu.VMEM((CHUNK, 128), F32),
            pltpu.VMEM((CHUNK, 128), F32),
            pltpu.SemaphoreType.DMA,
            pltpu.SemaphoreType.DMA,
        ],
    )
    def k(hsa_hbm, hsb_hbm, dest_hbm, outa_hbm, outb_hbm, idx_v, rowa_v,
          rowb_v, sema, semb):
        wid = lax.axis_index("s") * SC_CORES + lax.axis_index("c")
        pltpu.sync_copy(dest_hbm.at[wid], idx_v)
        base = wid * per_w
        for j in range(nchunks):
            ca = pltpu.async_copy(hsa_hbm.at[idx_v.at[j]], rowa_v, sema)
            cb = pltpu.async_copy(hsb_hbm.at[idx_v.at[j]], rowb_v, semb)
            ca.wait()
            cb.wait()
            pltpu.sync_copy(rowa_v, outa_hbm.at[pl.ds(base + j * CHUNK, CHUNK)])
            pltpu.sync_copy(rowb_v, outb_hbm.at[pl.ds(base + j * CHUNK, CHUNK)])

    return k(hsa, hsb, dest3)


def _head_body(hrawa_ref, hrawb_ref, gtop_ref, vdirT_ref,
               wr1az_ref, wvs_ref, wvc_ref, wvi_ref, br1z_ref,
               wr2_ref, br2_ref,
               sig_ref, rgb_ref, ssum_ref):
    i = pl.program_id(0)
    so = jnp.concatenate([hrawa_ref[...], hrawb_ref[...]],
                         axis=1) * gtop_ref[...]
    vt = vdirT_ref[...]
    t12 = jnp.concatenate([vt * (2.0 ** d) for d in range(NDF)], axis=0)
    s = jnp.sin(t12)
    c = jnp.cos(t12)
    u = (jnp.dot(so, wr1az_ref[...], preferred_element_type=F32)
         + lax.dot_general(s, wvs_ref[...], _DN0, preferred_element_type=F32)
         + lax.dot_general(c, wvc_ref[...], _DN0, preferred_element_type=F32)
         + lax.dot_general(vt, wvi_ref[...], _DN0, preferred_element_type=F32)
         + br1z_ref[...])
    z = u[:, 128:129]
    sig = jnp.maximum(z, 0.0) + jnp.log(1.0 + jnp.exp(-jnp.abs(z)))
    sig_ref[...] = sig
    hr = jnp.maximum(u[:, :128], 0.0)
    t = jnp.dot(hr, wr2_ref[...], preferred_element_type=F32) + br2_ref[...]
    rgb_ref[...] = 1.0 / (1.0 + jnp.exp(-t))

    @pl.when(i == 0)
    def _():
        ssum_ref[...] = jnp.zeros_like(ssum_ref)

    ssum_ref[...] += jnp.sum(sig, axis=0, keepdims=True)


def _heads(hrawa, hrawb, gtop, vdirT, wr1az, wvs, wvc, wvi, br1z, wr2p,
           br2p, n):
    bf = 1024
    return pl.pallas_call(
        _head_body,
        grid=(n // bf,),
        in_specs=[
            pl.BlockSpec((bf, 128), lambda i: (i, 0)),
            pl.BlockSpec((bf, 128), lambda i: (i, 0)),
            pl.BlockSpec((bf, 1), lambda i: (i, 0)),
            pl.BlockSpec((3, bf), lambda i: (0, i)),
            pl.BlockSpec((ENC, 256), lambda i: (0, 0)),
            pl.BlockSpec((3 * NDF, 256), lambda i: (0, 0)),
            pl.BlockSpec((3 * NDF, 256), lambda i: (0, 0)),
            pl.BlockSpec((3, 256), lambda i: (0, 0)),
            pl.BlockSpec((1, 256), lambda i: (0, 0)),
            pl.BlockSpec((128, 128), lambda i: (0, 0)),
            pl.BlockSpec((1, 128), lambda i: (0, 0)),
        ],
        out_specs=[
            pl.BlockSpec((bf, 1), lambda i: (i, 0)),
            pl.BlockSpec((bf, 128), lambda i: (i, 0)),
            pl.BlockSpec((1, 1), lambda i: (0, 0)),
        ],
        out_shape=[
            jax.ShapeDtypeStruct((n, 1), F32),
            jax.ShapeDtypeStruct((n, 128), F32),
            jax.ShapeDtypeStruct((1, 1), F32),
        ],
    )(hrawa, hrawb, gtop, vdirT, wr1az, wvs, wvc, wvi,
      br1z, wr2p, br2p)


def kernel(xyz, viewdir, shape_latent, texture_latent, temperature,
           W_enc, b_enc, W_g, b_g,
           We1, be1, We2, be2, We3, be3, We4, be4,
           W_sig, b_sig, W_r1, b_r1, W_r2, b_r2):
    nrays, nsamples, _ = xyz.shape
    n = nrays * nsamples
    npad = (n // T + E) * T

    xyzT = jnp.transpose(xyz, (2, 1, 0)).reshape(3, n)
    vdirT = jnp.transpose(viewdir, (2, 1, 0)).reshape(3, n)
    temp11 = temperature.reshape(1, 1)
    nsf = 3 * NXF
    wsin30 = W_enc[3:3 + nsf]
    wcos30 = W_enc[3 + nsf:3 + 2 * nsf]
    wid3 = W_enc[:3]

    ya, yb, gates, onehot, gtop, counts, gsum = _gating(
        xyzT, temp11, wsin30, wcos30, wid3, b_enc, W_g, b_g, n)

    cnt = counts.reshape(E)
    tile_cnt = jnp.ceil(cnt / T).astype(I32)
    tile_start = jnp.concatenate(
        [jnp.zeros((1,), I32), jnp.cumsum(tile_cnt)[:-1]])
    starts18 = (tile_start * T).astype(F32).reshape(1, E)
    nt = npad // T
    cum = jnp.cumsum(tile_cnt)
    tidx = jnp.arange(nt, dtype=I32)
    tile_eid = jnp.minimum(
        jnp.sum((tidx[:, None] >= cum[None, :]).astype(I32), axis=1),
        E - 1).astype(I32)

    dest = _dest(onehot, starts18, n)
    dest3 = dest.reshape(NWORK, (n // NWORK) // CHUNK, CHUNK)

    ysa, ysb = _dispatch_scatter(ya, yb, dest3, npad)
    hsa, hsb = _experts(tile_eid, ysa, ysb, We1, be1, We2, be2, We3, be3,
                        We4, be4, npad)
    hrawa, hrawb = _combine_gather(hsa, hsb, dest3, n)

    ncf = 3 * NDF
    wr1az = jnp.concatenate(
        [W_r1[:ENC], W_sig, jnp.zeros((ENC, 127), F32)], axis=1)
    wvs = jnp.zeros((ncf, 256), F32).at[:, :128].set(W_r1[ENC + 3:ENC + 3 + ncf])
    wvc = jnp.zeros((ncf, 256), F32).at[:, :128].set(W_r1[ENC + 3 + ncf:])
    wvi = jnp.zeros((3, 256), F32).at[:, :128].set(W_r1[ENC:ENC + 3])
    br1z = jnp.concatenate(
        [b_r1, b_sig, jnp.zeros((127,), F32)]).reshape(1, 256)
    wr2p = jnp.concatenate([W_r2, jnp.zeros((128, 125), F32)], axis=1)
    br2p = jnp.concatenate([b_r2, jnp.zeros((125,), F32)]).reshape(1, 128)

    sig, rgbp, ssum = _heads(hrawa, hrawb, gtop, vdirT, wr1az, wvs, wvc,
                             wvi, br1z, wr2p, br2p, n)

    sigmas = sig.reshape(nsamples, nrays, 1).transpose(1, 0, 2)
    rgbs = rgbp[:, :3].reshape(nsamples, nrays, 3).transpose(1, 0, 2)
    gates_soft_o = gates.reshape(nsamples, nrays, E).transpose(1, 0, 2)
    gates_hard_o = onehot.reshape(nsamples, nrays, E).transpose(1, 0, 2)
    mean_sigma = (ssum / n).reshape(1)
    num_pts = cnt
    aux_loss = E * jnp.sum((cnt / n) * (gsum.reshape(E) / n))
    return (sigmas, rgbs, gates_soft_o, gates_hard_o,
            mean_sigma, num_pts, aux_loss)


# speedup vs baseline: 1.4273x; 1.1108x over previous
import functools

import jax
import jax.numpy as jnp
import numpy as np
from jax import lax
from jax.experimental import pallas as pl
from jax.experimental.pallas import tpu as pltpu
from jax.experimental.pallas import tpu_sc as plsc

F32 = jnp.float32
I32 = jnp.int32

E = 8
ENC = 256
WID = 256
NXF = 10
NDF = 4
T = 512

SC_CORES = 2
SC_SUBCORES = 16
NWORK = SC_CORES * SC_SUBCORES
CHUNK = 128


BF16 = jnp.bfloat16


def _split3(a):
    ah = a.astype(BF16)
    al = (a - ah.astype(F32)).astype(BF16)
    return ah, al


def _dot3(a, b):
    ah, al = _split3(a)
    bh, bl = _split3(b)
    return (jnp.dot(ah, bh, preferred_element_type=F32)
            + jnp.dot(ah, bl, preferred_element_type=F32)
            + jnp.dot(al, bh, preferred_element_type=F32))


def _dot3_pre(ah, al, bh, bl):
    return (jnp.dot(ah, bh, preferred_element_type=F32)
            + jnp.dot(ah, bl, preferred_element_type=F32)
            + jnp.dot(al, bh, preferred_element_type=F32))


def _pe_matrix(degree, width):
    m = np.zeros((3, width), np.float32)
    half = width // 2
    for d in range(degree):
        for c in range(3):
            m[c, 3 * d + c] = 2.0 ** d
            m[c, half + 3 * d + c] = 2.0 ** d
    return jnp.asarray(m)


def _pe_sincos(x, mat):
    t = (x[:, 0:1] * mat[0:1, :] + x[:, 1:2] * mat[1:2, :]
         + x[:, 2:3] * mat[2:3, :])
    half = t.shape[1] // 2
    return jnp.sin(t[:, :half]), jnp.cos(t[:, half:])


_DN0 = (((0,), (0,)), ((), ()))


def _gating_body(temp_ref, xyzT_ref, wsin_ref, wcos_ref, wid_ref, benc_ref,
                 wg_ref, bg_ref,
                 ya_ref, yb_ref, gates_ref, onehot_ref, gtop_ref,
                 counts_ref, gsum_ref):
    i = pl.program_id(0)
    xt = xyzT_ref[...]
    t30 = jnp.concatenate([xt * (2.0 ** d) for d in range(NXF)], axis=0)
    s = jnp.sin(t30)
    c = jnp.cos(t30)
    y = (lax.dot_general(s, wsin_ref[...], _DN0, preferred_element_type=F32)
         + lax.dot_general(c, wcos_ref[...], _DN0, preferred_element_type=F32)
         + lax.dot_general(xt, wid_ref[...], _DN0, preferred_element_type=F32)
         + benc_ref[...])
    ya_ref[...] = y[:, :128]
    yb_ref[...] = y[:, 128:]
    logits = jnp.dot(y, wg_ref[...], preferred_element_type=F32) + bg_ref[...]
    lt = logits / temp_ref[0, 0]
    m = jnp.max(lt, axis=1, keepdims=True)
    ex = jnp.exp(lt - m)
    g = ex / jnp.sum(ex, axis=1, keepdims=True)
    gates_ref[...] = g
    li = lax.broadcasted_iota(I32, g.shape, 1)
    gm = jnp.max(g, axis=1, keepdims=True)
    am = jnp.min(jnp.where(g == gm, li, E), axis=1, keepdims=True)
    oh = (li == am).astype(F32)
    onehot_ref[...] = oh
    gtop_ref[...] = gm

    @pl.when(i == 0)
    def _():
        counts_ref[...] = jnp.zeros_like(counts_ref)
        gsum_ref[...] = jnp.zeros_like(gsum_ref)

    counts_ref[...] += jnp.sum(oh, axis=0, keepdims=True)
    gsum_ref[...] += jnp.sum(g, axis=0, keepdims=True)


def _gating(xyzT, temp11, wsin30, wcos30, wid3, b_enc, W_g, b_g, n):
    ba = 1024
    grid = (n // ba,)
    return pl.pallas_call(
        _gating_body,
        grid=grid,
        in_specs=[
            pl.BlockSpec(memory_space=pltpu.SMEM),
            pl.BlockSpec((3, ba), lambda i: (0, i)),
            pl.BlockSpec((3 * NXF, ENC), lambda i: (0, 0)),
            pl.BlockSpec((3 * NXF, ENC), lambda i: (0, 0)),
            pl.BlockSpec((3, ENC), lambda i: (0, 0)),
            pl.BlockSpec((1, ENC), lambda i: (0, 0)),
            pl.BlockSpec((ENC, E), lambda i: (0, 0)),
            pl.BlockSpec((1, E), lambda i: (0, 0)),
        ],
        out_specs=[
            pl.BlockSpec((ba, 128), lambda i: (i, 0)),
            pl.BlockSpec((ba, 128), lambda i: (i, 0)),
            pl.BlockSpec((ba, E), lambda i: (i, 0)),
            pl.BlockSpec((ba, E), lambda i: (i, 0)),
            pl.BlockSpec((ba, 1), lambda i: (i, 0)),
            pl.BlockSpec((1, E), lambda i: (0, 0)),
            pl.BlockSpec((1, E), lambda i: (0, 0)),
        ],
        out_shape=[
            jax.ShapeDtypeStruct((n, 128), F32),
            jax.ShapeDtypeStruct((n, 128), F32),
            jax.ShapeDtypeStruct((n, E), F32),
            jax.ShapeDtypeStruct((n, E), F32),
            jax.ShapeDtypeStruct((n, 1), F32),
            jax.ShapeDtypeStruct((1, E), F32),
            jax.ShapeDtypeStruct((1, E), F32),
        ],
    )(temp11, xyzT, wsin30, wcos30, wid3,
      b_enc.reshape(1, ENC), W_g, b_g.reshape(1, E))


def _dest_body(onehot_ref, starts_ref, ltri_ref, dest_ref, carry_ref):
    i = pl.program_id(0)

    @pl.when(i == 0)
    def _():
        carry_ref[...] = jnp.zeros_like(carry_ref)

    oh = onehot_ref[...]
    ranks = jnp.dot(ltri_ref[...], oh, preferred_element_type=F32,
                    precision=lax.Precision.DEFAULT)
    base = starts_ref[...] + carry_ref[...]
    destf = jnp.sum(oh * (base + ranks), axis=1, keepdims=True)
    dest_ref[...] = destf.astype(I32)
    carry_ref[...] += jnp.sum(oh, axis=0, keepdims=True)


def _dest(onehot, starts18, n):
    tb = 512
    r = np.arange(tb)
    ltri = jnp.asarray((r[:, None] > r[None, :]).astype(np.float32))
    return pl.pallas_call(
        _dest_body,
        grid=(n // tb,),
        in_specs=[
            pl.BlockSpec((tb, E), lambda i: (i, 0)),
            pl.BlockSpec((1, E), lambda i: (0, 0)),
            pl.BlockSpec((tb, tb), lambda i: (0, 0)),
        ],
        out_specs=pl.BlockSpec((tb, 1), lambda i: (i, 0)),
        out_shape=jax.ShapeDtypeStruct((n, 1), I32),
        scratch_shapes=[pltpu.VMEM((1, E), F32)],
    )(onehot, starts18, ltri)


def _dispatch_scatter(ya, yb, dest3, npad):
    n = ya.shape[0]
    per_w = n // NWORK
    nchunks = per_w // CHUNK
    mesh = plsc.VectorSubcoreMesh(core_axis_name="c", subcore_axis_name="s")

    @functools.partial(
        pl.kernel,
        mesh=mesh,
        out_type=[jax.ShapeDtypeStruct((npad, 128), F32),
                  jax.ShapeDtypeStruct((npad, 128), F32)],
        scratch_types=[
            pltpu.VMEM((nchunks, CHUNK), I32),
            pltpu.VMEM((CHUNK, 128), F32),
            pltpu.VMEM((CHUNK, 128), F32),
            pltpu.SemaphoreType.DMA,
            pltpu.SemaphoreType.DMA,
        ],
    )
    def k(ya_hbm, yb_hbm, dest_hbm, ysa_hbm, ysb_hbm, idx_v, rowa_v, rowb_v,
          sema, semb):
        wid = lax.axis_index("s") * SC_CORES + lax.axis_index("c")
        pltpu.sync_copy(dest_hbm.at[wid], idx_v)
        base = wid * per_w
        for j in range(nchunks):
            pltpu.sync_copy(ya_hbm.at[pl.ds(base + j * CHUNK, CHUNK)], rowa_v)
            pltpu.sync_copy(yb_hbm.at[pl.ds(base + j * CHUNK, CHUNK)], rowb_v)
            ca = pltpu.async_copy(rowa_v, ysa_hbm.at[idx_v.at[j]], sema)
            cb = pltpu.async_copy(rowb_v, ysb_hbm.at[idx_v.at[j]], semb)
            ca.wait()
            cb.wait()

    return k(ya, yb, dest3)


def _expert_body(eid_ref, ysa_ref, ysb_ref, w1_ref, b1_ref, w2_ref, b2_ref,
                 w3_ref, b3_ref, w4_ref, b4_ref, outa_ref, outb_ref):
    t = pl.program_id(0)
    e = eid_ref[t]
    a = jnp.concatenate([ysa_ref[...], ysb_ref[...]], axis=1)
    h = jnp.maximum(jnp.dot(a, w1_ref[e], preferred_element_type=F32) + b1_ref[e], 0.0)
    h = jnp.maximum(jnp.dot(h, w2_ref[e], preferred_element_type=F32) + b2_ref[e], 0.0)
    h = jnp.maximum(jnp.dot(h, w3_ref[e], preferred_element_type=F32) + b3_ref[e], 0.0)
    h = jnp.dot(h, w4_ref[e], preferred_element_type=F32) + b4_ref[e]
    outa_ref[...] = h[:, :128]
    outb_ref[...] = h[:, 128:]


def _experts(tile_eid, ysa, ysb, We1, be1, We2, be2, We3, be3, We4, be4,
             npad):
    nt = npad // T
    wspec = pl.BlockSpec((E, ENC, WID), lambda t, eid: (0, 0, 0))
    bspec = pl.BlockSpec((E, 1, WID), lambda t, eid: (0, 0, 0))
    grid_spec = pltpu.PrefetchScalarGridSpec(
        num_scalar_prefetch=1,
        grid=(nt,),
        in_specs=[
            pl.BlockSpec((T, 128), lambda t, eid: (t, 0)),
            pl.BlockSpec((T, 128), lambda t, eid: (t, 0)),
            wspec, bspec, wspec, bspec, wspec, bspec, wspec, bspec,
        ],
        out_specs=[pl.BlockSpec((T, 128), lambda t, eid: (t, 0)),
                   pl.BlockSpec((T, 128), lambda t, eid: (t, 0))],
    )
    return pl.pallas_call(
        _expert_body,
        grid_spec=grid_spec,
        out_shape=[jax.ShapeDtypeStruct((npad, 128), F32),
                   jax.ShapeDtypeStruct((npad, 128), F32)],
    )(tile_eid, ysa, ysb,
      We1, be1.reshape(E, 1, WID), We2, be2.reshape(E, 1, WID),
      We3, be3.reshape(E, 1, WID), We4, be4.reshape(E, 1, WID))


def _combine_gather(hsa, hsb, dest3, n):
    per_w = n // NWORK
    nchunks = per_w // CHUNK
    mesh = plsc.VectorSubcoreMesh(core_axis_name="c", subcore_axis_name="s")

    @functools.partial(
        pl.kernel,
        mesh=mesh,
        out_type=[jax.ShapeDtypeStruct((n, 128), F32),
                  jax.ShapeDtypeStruct((n, 128), F32)],
        scratch_types=[
            pltpu.VMEM((nchunks, CHUNK), I32),
            pltpu.VMEM((CHUNK, 128), F32),
            pltpu.VMEM((CHUNK, 128), F32),
            pltpu.SemaphoreType.DMA,
            pltpu.SemaphoreType.DMA,
        ],
    )
    def k(hsa_hbm, hsb_hbm, dest_hbm, outa_hbm, outb_hbm, idx_v, rowa_v,
          rowb_v, sema, semb):
        wid = lax.axis_index("s") * SC_CORES + lax.axis_index("c")
        pltpu.sync_copy(dest_hbm.at[wid], idx_v)
        base = wid * per_w
        for j in range(nchunks):
            ca = pltpu.async_copy(hsa_hbm.at[idx_v.at[j]], rowa_v, sema)
            cb = pltpu.async_copy(hsb_hbm.at[idx_v.at[j]], rowb_v, semb)
            ca.wait()
            cb.wait()
            pltpu.sync_copy(rowa_v, outa_hbm.at[pl.ds(base + j * CHUNK, CHUNK)])
            pltpu.sync_copy(rowb_v, outb_hbm.at[pl.ds(base + j * CHUNK, CHUNK)])

    return k(hsa, hsb, dest3)


def _head_body(hrawa_ref, hrawb_ref, gtop_ref, vdirT_ref,
               wr1az_ref, wvs_ref, wvc_ref, wvi_ref, br1z_ref,
               wr2_ref, br2_ref,
               sig_ref, rgb_ref, ssum_ref):
    i = pl.program_id(0)
    so = jnp.concatenate([hrawa_ref[...], hrawb_ref[...]],
                         axis=1) * gtop_ref[...]
    vt = vdirT_ref[...]
    t12 = jnp.concatenate([vt * (2.0 ** d) for d in range(NDF)], axis=0)
    s = jnp.sin(t12)
    c = jnp.cos(t12)
    u = (jnp.dot(so, wr1az_ref[...], preferred_element_type=F32)
         + lax.dot_general(s, wvs_ref[...], _DN0, preferred_element_type=F32)
         + lax.dot_general(c, wvc_ref[...], _DN0, preferred_element_type=F32)
         + lax.dot_general(vt, wvi_ref[...], _DN0, preferred_element_type=F32)
         + br1z_ref[...])
    z = u[:, 128:129]
    sig = jnp.maximum(z, 0.0) + jnp.log(1.0 + jnp.exp(-jnp.abs(z)))
    sig_ref[...] = sig
    hr = jnp.maximum(u[:, :128], 0.0)
    t = jnp.dot(hr, wr2_ref[...], preferred_element_type=F32) + br2_ref[...]
    rgb_ref[...] = 1.0 / (1.0 + jnp.exp(-t))

    @pl.when(i == 0)
    def _():
        ssum_ref[...] = jnp.zeros_like(ssum_ref)

    ssum_ref[...] += jnp.sum(sig, axis=0, keepdims=True)


def _heads(hrawa, hrawb, gtop, vdirT, wr1az, wvs, wvc, wvi, br1z, wr2p,
           br2p, n):
    bf = 1024
    return pl.pallas_call(
        _head_body,
        grid=(n // bf,),
        in_specs=[
            pl.BlockSpec((bf, 128), lambda i: (i, 0)),
            pl.BlockSpec((bf, 128), lambda i: (i, 0)),
            pl.BlockSpec((bf, 1), lambda i: (i, 0)),
            pl.BlockSpec((3, bf), lambda i: (0, i)),
            pl.BlockSpec((ENC, 256), lambda i: (0, 0)),
            pl.BlockSpec((3 * NDF, 256), lambda i: (0, 0)),
            pl.BlockSpec((3 * NDF, 256), lambda i: (0, 0)),
            pl.BlockSpec((3, 256), lambda i: (0, 0)),
            pl.BlockSpec((1, 256), lambda i: (0, 0)),
            pl.BlockSpec((128, 128), lambda i: (0, 0)),
            pl.BlockSpec((1, 128), lambda i: (0, 0)),
        ],
        out_specs=[
            pl.BlockSpec((bf, 1), lambda i: (i, 0)),
            pl.BlockSpec((bf, 128), lambda i: (i, 0)),
            pl.BlockSpec((1, 1), lambda i: (0, 0)),
        ],
        out_shape=[
            jax.ShapeDtypeStruct((n, 1), F32),
            jax.ShapeDtypeStruct((n, 128), F32),
            jax.ShapeDtypeStruct((1, 1), F32),
        ],
    )(hrawa, hrawb, gtop, vdirT, wr1az, wvs, wvc, wvi,
      br1z, wr2p, br2p)


def kernel(xyz, viewdir, shape_latent, texture_latent, temperature,
           W_enc, b_enc, W_g, b_g,
           We1, be1, We2, be2, We3, be3, We4, be4,
           W_sig, b_sig, W_r1, b_r1, W_r2, b_r2):
    nrays, nsamples, _ = xyz.shape
    n = nrays * nsamples
    npad = (n // T + E) * T

    xyzT = jnp.transpose(xyz, (2, 1, 0)).reshape(3, n)
    vdirT = jnp.transpose(viewdir, (2, 1, 0)).reshape(3, n)
    temp11 = temperature.reshape(1, 1)
    nsf = 3 * NXF
    wsin30 = W_enc[3:3 + nsf]
    wcos30 = W_enc[3 + nsf:3 + 2 * nsf]
    wid3 = W_enc[:3]

    ya, yb, gates, onehot, gtop, counts, gsum = _gating(
        xyzT, temp11, wsin30, wcos30, wid3, b_enc, W_g, b_g, n)

    cnt = counts.reshape(E)
    tile_cnt = jnp.ceil(cnt / T).astype(I32)
    tile_start = jnp.concatenate(
        [jnp.zeros((1,), I32), jnp.cumsum(tile_cnt)[:-1]])
    starts18 = (tile_start * T).astype(F32).reshape(1, E)
    nt = npad // T
    cum = jnp.cumsum(tile_cnt)
    tidx = jnp.arange(nt, dtype=I32)
    tile_eid = jnp.minimum(
        jnp.sum((tidx[:, None] >= cum[None, :]).astype(I32), axis=1),
        E - 1).astype(I32)

    dest = _dest(onehot, starts18, n)
    dest3 = dest.reshape(NWORK, (n // NWORK) // CHUNK, CHUNK)

    ysa, ysb = _dispatch_scatter(ya, yb, dest3, npad)
    hsa, hsb = _experts(tile_eid, ysa, ysb, We1, be1, We2, be2, We3, be3,
                        We4, be4, npad)
    hrawa, hrawb = _combine_gather(hsa, hsb, dest3, n)

    ncf = 3 * NDF
    wr1az = jnp.concatenate(
        [W_r1[:ENC], W_sig, jnp.zeros((ENC, 127), F32)], axis=1)
    wvs = jnp.zeros((ncf, 256), F32).at[:, :128].set(W_r1[ENC + 3:ENC + 3 + ncf])
    wvc = jnp.zeros((ncf, 256), F32).at[:, :128].set(W_r1[ENC + 3 + ncf:])
    wvi = jnp.zeros((3, 256), F32).at[:, :128].set(W_r1[ENC:ENC + 3])
    br1z = jnp.concatenate(
        [b_r1, b_sig, jnp.zeros((127,), F32)]).reshape(1, 256)
    wr2p = jnp.concatenate([W_r2, jnp.zeros((128, 125), F32)], axis=1)
    br2p = jnp.concatenate([b_r2, jnp.zeros((125,), F32)]).reshape(1, 128)

    sig, rgbp, ssum = _heads(hrawa, hrawb, gtop, vdirT, wr1az, wvs, wvc,
                             wvi, br1z, wr2p, br2p, n)

    sigmas = sig.reshape(nsamples, nrays, 1).transpose(1, 0, 2)
    rgbs = rgbp[:, :3].reshape(nsamples, nrays, 3).transpose(1, 0, 2)
    gates_soft_o = gates.reshape(nsamples, nrays, E).transpose(1, 0, 2)
    gates_hard_o = onehot.reshape(nsamples, nrays, E).transpose(1, 0, 2)
    mean_sigma = (ssum / n).reshape(1)
    num_pts = cnt
    aux_loss = E * jnp.sum((cnt / n) * (gsum.reshape(E) / n))
    return (sigmas, rgbs, gates_soft_o, gates_hard_o,
            mean_sigma, num_pts, aux_loss)


# speedup vs baseline: 1.5054x; 1.0547x over previous
import functools

import jax
import jax.numpy as jnp
import numpy as np
from jax import lax
from jax.experimental import pallas as pl
from jax.experimental.pallas import tpu as pltpu
from jax.experimental.pallas import tpu_sc as plsc

F32 = jnp.float32
I32 = jnp.int32

E = 8
ENC = 256
WID = 256
NXF = 10
NDF = 4
T = 1024

SC_CORES = 2
SC_SUBCORES = 16
NWORK = SC_CORES * SC_SUBCORES
CHUNK = 128


BF16 = jnp.bfloat16


def _split3(a):
    ah = a.astype(BF16)
    al = (a - ah.astype(F32)).astype(BF16)
    return ah, al


def _dot3(a, b):
    ah, al = _split3(a)
    bh, bl = _split3(b)
    return (jnp.dot(ah, bh, preferred_element_type=F32)
            + jnp.dot(ah, bl, preferred_element_type=F32)
            + jnp.dot(al, bh, preferred_element_type=F32))


def _dot3_pre(ah, al, bh, bl):
    return (jnp.dot(ah, bh, preferred_element_type=F32)
            + jnp.dot(ah, bl, preferred_element_type=F32)
            + jnp.dot(al, bh, preferred_element_type=F32))


def _pe_matrix(degree, width):
    m = np.zeros((3, width), np.float32)
    half = width // 2
    for d in range(degree):
        for c in range(3):
            m[c, 3 * d + c] = 2.0 ** d
            m[c, half + 3 * d + c] = 2.0 ** d
    return jnp.asarray(m)


def _pe_sincos(x, mat):
    t = (x[:, 0:1] * mat[0:1, :] + x[:, 1:2] * mat[1:2, :]
         + x[:, 2:3] * mat[2:3, :])
    half = t.shape[1] // 2
    return jnp.sin(t[:, :half]), jnp.cos(t[:, half:])


_DN0 = (((0,), (0,)), ((), ()))


def _gating_body(temp_ref, xyzT_ref, wsin_ref, wcos_ref, wid_ref, benc_ref,
                 wg_ref, bg_ref,
                 ya_ref, yb_ref, gates_ref, onehot_ref, gtop_ref,
                 counts_ref, gsum_ref):
    i = pl.program_id(0)
    xt = xyzT_ref[...]
    t30 = jnp.concatenate([xt * (2.0 ** d) for d in range(NXF)], axis=0)
    s = jnp.sin(t30)
    c = jnp.cos(t30)
    y = (lax.dot_general(s, wsin_ref[...], _DN0, preferred_element_type=F32)
         + lax.dot_general(c, wcos_ref[...], _DN0, preferred_element_type=F32)
         + lax.dot_general(xt, wid_ref[...], _DN0, preferred_element_type=F32)
         + benc_ref[...])
    ya_ref[...] = y[:, :128]
    yb_ref[...] = y[:, 128:]
    logits = jnp.dot(y, wg_ref[...], preferred_element_type=F32) + bg_ref[...]
    lt = logits / temp_ref[0, 0]
    m = jnp.max(lt, axis=1, keepdims=True)
    ex = jnp.exp(lt - m)
    g = ex / jnp.sum(ex, axis=1, keepdims=True)
    gates_ref[...] = g
    li = lax.broadcasted_iota(I32, g.shape, 1)
    gm = jnp.max(g, axis=1, keepdims=True)
    am = jnp.min(jnp.where(g == gm, li, E), axis=1, keepdims=True)
    oh = (li == am).astype(F32)
    onehot_ref[...] = oh
    gtop_ref[...] = gm

    @pl.when(i == 0)
    def _():
        counts_ref[...] = jnp.zeros_like(counts_ref)
        gsum_ref[...] = jnp.zeros_like(gsum_ref)

    counts_ref[...] += jnp.sum(oh, axis=0, keepdims=True)
    gsum_ref[...] += jnp.sum(g, axis=0, keepdims=True)


def _gating(xyzT, temp11, wsin30, wcos30, wid3, b_enc, W_g, b_g, n):
    ba = 1024
    grid = (n // ba,)
    return pl.pallas_call(
        _gating_body,
        grid=grid,
        in_specs=[
            pl.BlockSpec(memory_space=pltpu.SMEM),
            pl.BlockSpec((3, ba), lambda i: (0, i)),
            pl.BlockSpec((3 * NXF, ENC), lambda i: (0, 0)),
            pl.BlockSpec((3 * NXF, ENC), lambda i: (0, 0)),
            pl.BlockSpec((3, ENC), lambda i: (0, 0)),
            pl.BlockSpec((1, ENC), lambda i: (0, 0)),
            pl.BlockSpec((ENC, E), lambda i: (0, 0)),
            pl.BlockSpec((1, E), lambda i: (0, 0)),
        ],
        out_specs=[
            pl.BlockSpec((ba, 128), lambda i: (i, 0)),
            pl.BlockSpec((ba, 128), lambda i: (i, 0)),
            pl.BlockSpec((ba, E), lambda i: (i, 0)),
            pl.BlockSpec((ba, E), lambda i: (i, 0)),
            pl.BlockSpec((ba, 1), lambda i: (i, 0)),
            pl.BlockSpec((1, E), lambda i: (0, 0)),
            pl.BlockSpec((1, E), lambda i: (0, 0)),
        ],
        out_shape=[
            jax.ShapeDtypeStruct((n, 128), F32),
            jax.ShapeDtypeStruct((n, 128), F32),
            jax.ShapeDtypeStruct((n, E), F32),
            jax.ShapeDtypeStruct((n, E), F32),
            jax.ShapeDtypeStruct((n, 1), F32),
            jax.ShapeDtypeStruct((1, E), F32),
            jax.ShapeDtypeStruct((1, E), F32),
        ],
    )(temp11, xyzT, wsin30, wcos30, wid3,
      b_enc.reshape(1, ENC), W_g, b_g.reshape(1, E))


def _dest_body(onehot_ref, starts_ref, ltri_ref, dest_ref, carry_ref):
    i = pl.program_id(0)

    @pl.when(i == 0)
    def _():
        carry_ref[...] = jnp.zeros_like(carry_ref)

    oh = onehot_ref[...]
    ranks = jnp.dot(ltri_ref[...], oh, preferred_element_type=F32,
                    precision=lax.Precision.DEFAULT)
    base = starts_ref[...] + carry_ref[...]
    destf = jnp.sum(oh * (base + ranks), axis=1, keepdims=True)
    dest_ref[...] = destf.astype(I32)
    carry_ref[...] += jnp.sum(oh, axis=0, keepdims=True)


def _dest(onehot, starts18, n):
    tb = 512
    r = np.arange(tb)
    ltri = jnp.asarray((r[:, None] > r[None, :]).astype(np.float32))
    return pl.pallas_call(
        _dest_body,
        grid=(n // tb,),
        in_specs=[
            pl.BlockSpec((tb, E), lambda i: (i, 0)),
            pl.BlockSpec((1, E), lambda i: (0, 0)),
            pl.BlockSpec((tb, tb), lambda i: (0, 0)),
        ],
        out_specs=pl.BlockSpec((tb, 1), lambda i: (i, 0)),
        out_shape=jax.ShapeDtypeStruct((n, 1), I32),
        scratch_shapes=[pltpu.VMEM((1, E), F32)],
    )(onehot, starts18, ltri)


def _dispatch_scatter(ya, yb, dest3, npad):
    n = ya.shape[0]
    per_w = n // NWORK
    nchunks = per_w // CHUNK
    mesh = plsc.VectorSubcoreMesh(core_axis_name="c", subcore_axis_name="s")

    @functools.partial(
        pl.kernel,
        mesh=mesh,
        out_type=[jax.ShapeDtypeStruct((npad, 128), F32),
                  jax.ShapeDtypeStruct((npad, 128), F32)],
        scratch_types=[
            pltpu.VMEM((nchunks, CHUNK), I32),
            pltpu.VMEM((CHUNK, 128), F32),
            pltpu.VMEM((CHUNK, 128), F32),
            pltpu.SemaphoreType.DMA,
            pltpu.SemaphoreType.DMA,
        ],
    )
    def k(ya_hbm, yb_hbm, dest_hbm, ysa_hbm, ysb_hbm, idx_v, rowa_v, rowb_v,
          sema, semb):
        wid = lax.axis_index("s") * SC_CORES + lax.axis_index("c")
        pltpu.sync_copy(dest_hbm.at[wid], idx_v)
        base = wid * per_w
        for j in range(nchunks):
            pltpu.sync_copy(ya_hbm.at[pl.ds(base + j * CHUNK, CHUNK)], rowa_v)
            pltpu.sync_copy(yb_hbm.at[pl.ds(base + j * CHUNK, CHUNK)], rowb_v)
            ca = pltpu.async_copy(rowa_v, ysa_hbm.at[idx_v.at[j]], sema)
            cb = pltpu.async_copy(rowb_v, ysb_hbm.at[idx_v.at[j]], semb)
            ca.wait()
            cb.wait()

    return k(ya, yb, dest3)


def _expert_body(eid_ref, ysa_ref, ysb_ref, w1_ref, b1_ref, w2_ref, b2_ref,
                 w3_ref, b3_ref, w4_ref, b4_ref, outa_ref, outb_ref):
    t = pl.program_id(0)
    e = eid_ref[t]
    a = jnp.concatenate([ysa_ref[...], ysb_ref[...]], axis=1)
    h = jnp.maximum(jnp.dot(a, w1_ref[e], preferred_element_type=F32) + b1_ref[e], 0.0)
    h = jnp.maximum(jnp.dot(h, w2_ref[e], preferred_element_type=F32) + b2_ref[e], 0.0)
    h = jnp.maximum(jnp.dot(h, w3_ref[e], preferred_element_type=F32) + b3_ref[e], 0.0)
    h = jnp.dot(h, w4_ref[e], preferred_element_type=F32) + b4_ref[e]
    outa_ref[...] = h[:, :128]
    outb_ref[...] = h[:, 128:]


def _experts(tile_eid, ysa, ysb, We1, be1, We2, be2, We3, be3, We4, be4,
             npad):
    nt = npad // T
    wspec = pl.BlockSpec((E, ENC, WID), lambda t, eid: (0, 0, 0))
    bspec = pl.BlockSpec((E, 1, WID), lambda t, eid: (0, 0, 0))
    grid_spec = pltpu.PrefetchScalarGridSpec(
        num_scalar_prefetch=1,
        grid=(nt,),
        in_specs=[
            pl.BlockSpec((T, 128), lambda t, eid: (t, 0)),
            pl.BlockSpec((T, 128), lambda t, eid: (t, 0)),
            wspec, bspec, wspec, bspec, wspec, bspec, wspec, bspec,
        ],
        out_specs=[pl.BlockSpec((T, 128), lambda t, eid: (t, 0)),
                   pl.BlockSpec((T, 128), lambda t, eid: (t, 0))],
    )
    return pl.pallas_call(
        _expert_body,
        grid_spec=grid_spec,
        out_shape=[jax.ShapeDtypeStruct((npad, 128), F32),
                   jax.ShapeDtypeStruct((npad, 128), F32)],
    )(tile_eid, ysa, ysb,
      We1, be1.reshape(E, 1, WID), We2, be2.reshape(E, 1, WID),
      We3, be3.reshape(E, 1, WID), We4, be4.reshape(E, 1, WID))


def _combine_gather(hsa, hsb, dest3, n):
    per_w = n // NWORK
    nchunks = per_w // CHUNK
    mesh = plsc.VectorSubcoreMesh(core_axis_name="c", subcore_axis_name="s")

    @functools.partial(
        pl.kernel,
        mesh=mesh,
        out_type=[jax.ShapeDtypeStruct((n, 128), F32),
                  jax.ShapeDtypeStruct((n, 128), F32)],
        scratch_types=[
            pltpu.VMEM((nchunks, CHUNK), I32),
            pltpu.VMEM((CHUNK, 128), F32),
            pltpu.VMEM((CHUNK, 128), F32),
            pltpu.SemaphoreType.DMA,
            pltpu.SemaphoreType.DMA,
        ],
    )
    def k(hsa_hbm, hsb_hbm, dest_hbm, outa_hbm, outb_hbm, idx_v, rowa_v,
          rowb_v, sema, semb):
        wid = lax.axis_index("s") * SC_CORES + lax.axis_index("c")
        pltpu.sync_copy(dest_hbm.at[wid], idx_v)
        base = wid * per_w
        for j in range(nchunks):
            ca = pltpu.async_copy(hsa_hbm.at[idx_v.at[j]], rowa_v, sema)
            cb = pltpu.async_copy(hsb_hbm.at[idx_v.at[j]], rowb_v, semb)
            ca.wait()
            cb.wait()
            pltpu.sync_copy(rowa_v, outa_hbm.at[pl.ds(base + j * CHUNK, CHUNK)])
            pltpu.sync_copy(rowb_v, outb_hbm.at[pl.ds(base + j * CHUNK, CHUNK)])

    return k(hsa, hsb, dest3)


def _head_body(hrawa_ref, hrawb_ref, gtop_ref, vdirT_ref,
               wr1az_ref, wvs_ref, wvc_ref, wvi_ref, br1z_ref,
               wr2_ref, br2_ref,
               sig_ref, rgb_ref, ssum_ref):
    i = pl.program_id(0)
    so = jnp.concatenate([hrawa_ref[...], hrawb_ref[...]],
                         axis=1) * gtop_ref[...]
    vt = vdirT_ref[...]
    t12 = jnp.concatenate([vt * (2.0 ** d) for d in range(NDF)], axis=0)
    s = jnp.sin(t12)
    c = jnp.cos(t12)
    u = (jnp.dot(so, wr1az_ref[...], preferred_element_type=F32)
         + lax.dot_general(s, wvs_ref[...], _DN0, preferred_element_type=F32)
         + lax.dot_general(c, wvc_ref[...], _DN0, preferred_element_type=F32)
         + lax.dot_general(vt, wvi_ref[...], _DN0, preferred_element_type=F32)
         + br1z_ref[...])
    z = u[:, 128:129]
    sig = jnp.maximum(z, 0.0) + jnp.log(1.0 + jnp.exp(-jnp.abs(z)))
    sig_ref[...] = sig
    hr = jnp.maximum(u[:, :128], 0.0)
    t = jnp.dot(hr, wr2_ref[...], preferred_element_type=F32) + br2_ref[...]
    rgb_ref[...] = 1.0 / (1.0 + jnp.exp(-t))

    @pl.when(i == 0)
    def _():
        ssum_ref[...] = jnp.zeros_like(ssum_ref)

    ssum_ref[...] += jnp.sum(sig, axis=0, keepdims=True)


def _heads(hrawa, hrawb, gtop, vdirT, wr1az, wvs, wvc, wvi, br1z, wr2p,
           br2p, n):
    bf = 1024
    return pl.pallas_call(
        _head_body,
        grid=(n // bf,),
        in_specs=[
            pl.BlockSpec((bf, 128), lambda i: (i, 0)),
            pl.BlockSpec((bf, 128), lambda i: (i, 0)),
            pl.BlockSpec((bf, 1), lambda i: (i, 0)),
            pl.BlockSpec((3, bf), lambda i: (0, i)),
            pl.BlockSpec((ENC, 256), lambda i: (0, 0)),
            pl.BlockSpec((3 * NDF, 256), lambda i: (0, 0)),
            pl.BlockSpec((3 * NDF, 256), lambda i: (0, 0)),
            pl.BlockSpec((3, 256), lambda i: (0, 0)),
            pl.BlockSpec((1, 256), lambda i: (0, 0)),
            pl.BlockSpec((128, 128), lambda i: (0, 0)),
            pl.BlockSpec((1, 128), lambda i: (0, 0)),
        ],
        out_specs=[
            pl.BlockSpec((bf, 1), lambda i: (i, 0)),
            pl.BlockSpec((bf, 128), lambda i: (i, 0)),
            pl.BlockSpec((1, 1), lambda i: (0, 0)),
        ],
        out_shape=[
            jax.ShapeDtypeStruct((n, 1), F32),
            jax.ShapeDtypeStruct((n, 128), F32),
            jax.ShapeDtypeStruct((1, 1), F32),
        ],
    )(hrawa, hrawb, gtop, vdirT, wr1az, wvs, wvc, wvi,
      br1z, wr2p, br2p)


def kernel(xyz, viewdir, shape_latent, texture_latent, temperature,
           W_enc, b_enc, W_g, b_g,
           We1, be1, We2, be2, We3, be3, We4, be4,
           W_sig, b_sig, W_r1, b_r1, W_r2, b_r2):
    nrays, nsamples, _ = xyz.shape
    n = nrays * nsamples
    npad = (n // T + E) * T

    xyzT = jnp.transpose(xyz, (2, 1, 0)).reshape(3, n)
    vdirT = jnp.transpose(viewdir, (2, 1, 0)).reshape(3, n)
    temp11 = temperature.reshape(1, 1)
    nsf = 3 * NXF
    wsin30 = W_enc[3:3 + nsf]
    wcos30 = W_enc[3 + nsf:3 + 2 * nsf]
    wid3 = W_enc[:3]

    ya, yb, gates, onehot, gtop, counts, gsum = _gating(
        xyzT, temp11, wsin30, wcos30, wid3, b_enc, W_g, b_g, n)

    cnt = counts.reshape(E)
    tile_cnt = jnp.ceil(cnt / T).astype(I32)
    tile_start = jnp.concatenate(
        [jnp.zeros((1,), I32), jnp.cumsum(tile_cnt)[:-1]])
    starts18 = (tile_start * T).astype(F32).reshape(1, E)
    nt = npad // T
    cum = jnp.cumsum(tile_cnt)
    tidx = jnp.arange(nt, dtype=I32)
    tile_eid = jnp.minimum(
        jnp.sum((tidx[:, None] >= cum[None, :]).astype(I32), axis=1),
        E - 1).astype(I32)

    dest = _dest(onehot, starts18, n)
    dest3 = dest.reshape(NWORK, (n // NWORK) // CHUNK, CHUNK)

    ysa, ysb = _dispatch_scatter(ya, yb, dest3, npad)
    hsa, hsb = _experts(tile_eid, ysa, ysb, We1, be1, We2, be2, We3, be3,
                        We4, be4, npad)
    hrawa, hrawb = _combine_gather(hsa, hsb, dest3, n)

    ncf = 3 * NDF
    wr1az = jnp.concatenate(
        [W_r1[:ENC], W_sig, jnp.zeros((ENC, 127), F32)], axis=1)
    wvs = jnp.zeros((ncf, 256), F32).at[:, :128].set(W_r1[ENC + 3:ENC + 3 + ncf])
    wvc = jnp.zeros((ncf, 256), F32).at[:, :128].set(W_r1[ENC + 3 + ncf:])
    wvi = jnp.zeros((3, 256), F32).at[:, :128].set(W_r1[ENC:ENC + 3])
    br1z = jnp.concatenate(
        [b_r1, b_sig, jnp.zeros((127,), F32)]).reshape(1, 256)
    wr2p = jnp.concatenate([W_r2, jnp.zeros((128, 125), F32)], axis=1)
    br2p = jnp.concatenate([b_r2, jnp.zeros((125,), F32)]).reshape(1, 128)

    sig, rgbp, ssum = _heads(hrawa, hrawb, gtop, vdirT, wr1az, wvs, wvc,
                             wvi, br1z, wr2p, br2p, n)

    sigmas = sig.reshape(nsamples, nrays, 1).transpose(1, 0, 2)
    rgbs = rgbp[:, :3].reshape(nsamples, nrays, 3).transpose(1, 0, 2)
    gates_soft_o = gates.reshape(nsamples, nrays, E).transpose(1, 0, 2)
    gates_hard_o = onehot.reshape(nsamples, nrays, E).transpose(1, 0, 2)
    mean_sigma = (ssum / n).reshape(1)
    num_pts = cnt
    aux_loss = E * jnp.sum((cnt / n) * (gsum.reshape(E) / n))
    return (sigmas, rgbs, gates_soft_o, gates_hard_o,
            mean_sigma, num_pts, aux_loss)


# speedup vs baseline: 1.5252x; 1.0132x over previous
import functools

import jax
import jax.numpy as jnp
import numpy as np
from jax import lax
from jax.experimental import pallas as pl
from jax.experimental.pallas import tpu as pltpu
from jax.experimental.pallas import tpu_sc as plsc

F32 = jnp.float32
I32 = jnp.int32

E = 8
ENC = 256
WID = 256
NXF = 10
NDF = 4
T = 2048

SC_CORES = 2
SC_SUBCORES = 16
NWORK = SC_CORES * SC_SUBCORES
CHUNK = 128


BF16 = jnp.bfloat16


def _split3(a):
    ah = a.astype(BF16)
    al = (a - ah.astype(F32)).astype(BF16)
    return ah, al


def _dot3(a, b):
    ah, al = _split3(a)
    bh, bl = _split3(b)
    return (jnp.dot(ah, bh, preferred_element_type=F32)
            + jnp.dot(ah, bl, preferred_element_type=F32)
            + jnp.dot(al, bh, preferred_element_type=F32))


def _dot3_pre(ah, al, bh, bl):
    return (jnp.dot(ah, bh, preferred_element_type=F32)
            + jnp.dot(ah, bl, preferred_element_type=F32)
            + jnp.dot(al, bh, preferred_element_type=F32))


def _pe_matrix(degree, width):
    m = np.zeros((3, width), np.float32)
    half = width // 2
    for d in range(degree):
        for c in range(3):
            m[c, 3 * d + c] = 2.0 ** d
            m[c, half + 3 * d + c] = 2.0 ** d
    return jnp.asarray(m)


def _pe_sincos(x, mat):
    t = (x[:, 0:1] * mat[0:1, :] + x[:, 1:2] * mat[1:2, :]
         + x[:, 2:3] * mat[2:3, :])
    half = t.shape[1] // 2
    return jnp.sin(t[:, :half]), jnp.cos(t[:, half:])


_DN0 = (((0,), (0,)), ((), ()))


def _gating_body(temp_ref, xyzT_ref, wsin_ref, wcos_ref, wid_ref, benc_ref,
                 wg_ref, bg_ref,
                 ya_ref, yb_ref, gates_ref, onehot_ref, gtop_ref,
                 counts_ref, gsum_ref):
    i = pl.program_id(0)
    xt = xyzT_ref[...]
    t30 = jnp.concatenate([xt * (2.0 ** d) for d in range(NXF)], axis=0)
    s = jnp.sin(t30)
    c = jnp.cos(t30)
    y = (lax.dot_general(s, wsin_ref[...], _DN0, preferred_element_type=F32)
         + lax.dot_general(c, wcos_ref[...], _DN0, preferred_element_type=F32)
         + lax.dot_general(xt, wid_ref[...], _DN0, preferred_element_type=F32)
         + benc_ref[...])
    ya_ref[...] = y[:, :128]
    yb_ref[...] = y[:, 128:]
    logits = jnp.dot(y, wg_ref[...], preferred_element_type=F32) + bg_ref[...]
    lt = logits / temp_ref[0, 0]
    m = jnp.max(lt, axis=1, keepdims=True)
    ex = jnp.exp(lt - m)
    g = ex / jnp.sum(ex, axis=1, keepdims=True)
    gates_ref[...] = g
    li = lax.broadcasted_iota(I32, g.shape, 1)
    gm = jnp.max(g, axis=1, keepdims=True)
    am = jnp.min(jnp.where(g == gm, li, E), axis=1, keepdims=True)
    oh = (li == am).astype(F32)
    onehot_ref[...] = oh
    gtop_ref[...] = gm

    @pl.when(i == 0)
    def _():
        counts_ref[...] = jnp.zeros_like(counts_ref)
        gsum_ref[...] = jnp.zeros_like(gsum_ref)

    counts_ref[...] += jnp.sum(oh, axis=0, keepdims=True)
    gsum_ref[...] += jnp.sum(g, axis=0, keepdims=True)


def _gating(xyzT, temp11, wsin30, wcos30, wid3, b_enc, W_g, b_g, n):
    ba = 1024
    grid = (n // ba,)
    return pl.pallas_call(
        _gating_body,
        grid=grid,
        in_specs=[
            pl.BlockSpec(memory_space=pltpu.SMEM),
            pl.BlockSpec((3, ba), lambda i: (0, i)),
            pl.BlockSpec((3 * NXF, ENC), lambda i: (0, 0)),
            pl.BlockSpec((3 * NXF, ENC), lambda i: (0, 0)),
            pl.BlockSpec((3, ENC), lambda i: (0, 0)),
            pl.BlockSpec((1, ENC), lambda i: (0, 0)),
            pl.BlockSpec((ENC, E), lambda i: (0, 0)),
            pl.BlockSpec((1, E), lambda i: (0, 0)),
        ],
        out_specs=[
            pl.BlockSpec((ba, 128), lambda i: (i, 0)),
            pl.BlockSpec((ba, 128), lambda i: (i, 0)),
            pl.BlockSpec((ba, E), lambda i: (i, 0)),
            pl.BlockSpec((ba, E), lambda i: (i, 0)),
            pl.BlockSpec((ba, 1), lambda i: (i, 0)),
            pl.BlockSpec((1, E), lambda i: (0, 0)),
            pl.BlockSpec((1, E), lambda i: (0, 0)),
        ],
        out_shape=[
            jax.ShapeDtypeStruct((n, 128), F32),
            jax.ShapeDtypeStruct((n, 128), F32),
            jax.ShapeDtypeStruct((n, E), F32),
            jax.ShapeDtypeStruct((n, E), F32),
            jax.ShapeDtypeStruct((n, 1), F32),
            jax.ShapeDtypeStruct((1, E), F32),
            jax.ShapeDtypeStruct((1, E), F32),
        ],
    )(temp11, xyzT, wsin30, wcos30, wid3,
      b_enc.reshape(1, ENC), W_g, b_g.reshape(1, E))


def _dest_body(onehot_ref, starts_ref, ltri_ref, dest_ref, carry_ref):
    i = pl.program_id(0)

    @pl.when(i == 0)
    def _():
        carry_ref[...] = jnp.zeros_like(carry_ref)

    oh = onehot_ref[...]
    ranks = jnp.dot(ltri_ref[...], oh, preferred_element_type=F32,
                    precision=lax.Precision.DEFAULT)
    base = starts_ref[...] + carry_ref[...]
    destf = jnp.sum(oh * (base + ranks), axis=1, keepdims=True)
    dest_ref[...] = destf.astype(I32)
    carry_ref[...] += jnp.sum(oh, axis=0, keepdims=True)


def _dest(onehot, starts18, n):
    tb = 512
    r = np.arange(tb)
    ltri = jnp.asarray((r[:, None] > r[None, :]).astype(np.float32))
    return pl.pallas_call(
        _dest_body,
        grid=(n // tb,),
        in_specs=[
            pl.BlockSpec((tb, E), lambda i: (i, 0)),
            pl.BlockSpec((1, E), lambda i: (0, 0)),
            pl.BlockSpec((tb, tb), lambda i: (0, 0)),
        ],
        out_specs=pl.BlockSpec((tb, 1), lambda i: (i, 0)),
        out_shape=jax.ShapeDtypeStruct((n, 1), I32),
        scratch_shapes=[pltpu.VMEM((1, E), F32)],
    )(onehot, starts18, ltri)


def _dispatch_scatter(ya, yb, dest3, npad):
    n = ya.shape[0]
    per_w = n // NWORK
    nchunks = per_w // CHUNK
    mesh = plsc.VectorSubcoreMesh(core_axis_name="c", subcore_axis_name="s")

    @functools.partial(
        pl.kernel,
        mesh=mesh,
        out_type=[jax.ShapeDtypeStruct((npad, 128), F32),
                  jax.ShapeDtypeStruct((npad, 128), F32)],
        scratch_types=[
            pltpu.VMEM((nchunks, CHUNK), I32),
            pltpu.VMEM((CHUNK, 128), F32),
            pltpu.VMEM((CHUNK, 128), F32),
            pltpu.SemaphoreType.DMA,
            pltpu.SemaphoreType.DMA,
        ],
    )
    def k(ya_hbm, yb_hbm, dest_hbm, ysa_hbm, ysb_hbm, idx_v, rowa_v, rowb_v,
          sema, semb):
        wid = lax.axis_index("s") * SC_CORES + lax.axis_index("c")
        pltpu.sync_copy(dest_hbm.at[wid], idx_v)
        base = wid * per_w
        for j in range(nchunks):
            pltpu.sync_copy(ya_hbm.at[pl.ds(base + j * CHUNK, CHUNK)], rowa_v)
            pltpu.sync_copy(yb_hbm.at[pl.ds(base + j * CHUNK, CHUNK)], rowb_v)
            ca = pltpu.async_copy(rowa_v, ysa_hbm.at[idx_v.at[j]], sema)
            cb = pltpu.async_copy(rowb_v, ysb_hbm.at[idx_v.at[j]], semb)
            ca.wait()
            cb.wait()

    return k(ya, yb, dest3)


def _expert_body(eid_ref, ysa_ref, ysb_ref, w1_ref, b1_ref, w2_ref, b2_ref,
                 w3_ref, b3_ref, w4_ref, b4_ref, outa_ref, outb_ref):
    t = pl.program_id(0)
    e = eid_ref[t]
    a = jnp.concatenate([ysa_ref[...], ysb_ref[...]], axis=1)
    h = jnp.maximum(jnp.dot(a, w1_ref[e], preferred_element_type=F32) + b1_ref[e], 0.0)
    h = jnp.maximum(jnp.dot(h, w2_ref[e], preferred_element_type=F32) + b2_ref[e], 0.0)
    h = jnp.maximum(jnp.dot(h, w3_ref[e], preferred_element_type=F32) + b3_ref[e], 0.0)
    h = jnp.dot(h, w4_ref[e], preferred_element_type=F32) + b4_ref[e]
    outa_ref[...] = h[:, :128]
    outb_ref[...] = h[:, 128:]


def _experts(tile_eid, ysa, ysb, We1, be1, We2, be2, We3, be3, We4, be4,
             npad):
    nt = npad // T
    wspec = pl.BlockSpec((E, ENC, WID), lambda t, eid: (0, 0, 0))
    bspec = pl.BlockSpec((E, 1, WID), lambda t, eid: (0, 0, 0))
    grid_spec = pltpu.PrefetchScalarGridSpec(
        num_scalar_prefetch=1,
        grid=(nt,),
        in_specs=[
            pl.BlockSpec((T, 128), lambda t, eid: (t, 0)),
            pl.BlockSpec((T, 128), lambda t, eid: (t, 0)),
            wspec, bspec, wspec, bspec, wspec, bspec, wspec, bspec,
        ],
        out_specs=[pl.BlockSpec((T, 128), lambda t, eid: (t, 0)),
                   pl.BlockSpec((T, 128), lambda t, eid: (t, 0))],
    )
    return pl.pallas_call(
        _expert_body,
        grid_spec=grid_spec,
        out_shape=[jax.ShapeDtypeStruct((npad, 128), F32),
                   jax.ShapeDtypeStruct((npad, 128), F32)],
    )(tile_eid, ysa, ysb,
      We1, be1.reshape(E, 1, WID), We2, be2.reshape(E, 1, WID),
      We3, be3.reshape(E, 1, WID), We4, be4.reshape(E, 1, WID))


def _combine_gather(hsa, hsb, dest3, n):
    per_w = n // NWORK
    nchunks = per_w // CHUNK
    mesh = plsc.VectorSubcoreMesh(core_axis_name="c", subcore_axis_name="s")

    @functools.partial(
        pl.kernel,
        mesh=mesh,
        out_type=[jax.ShapeDtypeStruct((n, 128), F32),
                  jax.ShapeDtypeStruct((n, 128), F32)],
        scratch_types=[
            pltpu.VMEM((nchunks, CHUNK), I32),
            pltpu.VMEM((CHUNK, 128), F32),
            pltpu.VMEM((CHUNK, 128), F32),
            pltpu.SemaphoreType.DMA,
            pltpu.SemaphoreType.DMA,
        ],
    )
    def k(hsa_hbm, hsb_hbm, dest_hbm, outa_hbm, outb_hbm, idx_v, rowa_v,
          rowb_v, sema, semb):
        wid = lax.axis_index("s") * SC_CORES + lax.axis_index("c")
        pltpu.sync_copy(dest_hbm.at[wid], idx_v)
        base = wid * per_w
        for j in range(nchunks):
            ca = pltpu.async_copy(hsa_hbm.at[idx_v.at[j]], rowa_v, sema)
            cb = pltpu.async_copy(hsb_hbm.at[idx_v.at[j]], rowb_v, semb)
            ca.wait()
            cb.wait()
            pltpu.sync_copy(rowa_v, outa_hbm.at[pl.ds(base + j * CHUNK, CHUNK)])
            pltpu.sync_copy(rowb_v, outb_hbm.at[pl.ds(base + j * CHUNK, CHUNK)])

    return k(hsa, hsb, dest3)


def _head_body(hrawa_ref, hrawb_ref, gtop_ref, vdirT_ref,
               wr1az_ref, wvs_ref, wvc_ref, wvi_ref, br1z_ref,
               wr2_ref, br2_ref,
               sig_ref, rgb_ref, ssum_ref):
    i = pl.program_id(0)
    so = jnp.concatenate([hrawa_ref[...], hrawb_ref[...]],
                         axis=1) * gtop_ref[...]
    vt = vdirT_ref[...]
    t12 = jnp.concatenate([vt * (2.0 ** d) for d in range(NDF)], axis=0)
    s = jnp.sin(t12)
    c = jnp.cos(t12)
    u = (jnp.dot(so, wr1az_ref[...], preferred_element_type=F32)
         + lax.dot_general(s, wvs_ref[...], _DN0, preferred_element_type=F32)
         + lax.dot_general(c, wvc_ref[...], _DN0, preferred_element_type=F32)
         + lax.dot_general(vt, wvi_ref[...], _DN0, preferred_element_type=F32)
         + br1z_ref[...])
    z = u[:, 128:129]
    sig = jnp.maximum(z, 0.0) + jnp.log(1.0 + jnp.exp(-jnp.abs(z)))
    sig_ref[...] = sig
    hr = jnp.maximum(u[:, :128], 0.0)
    t = jnp.dot(hr, wr2_ref[...], preferred_element_type=F32) + br2_ref[...]
    rgb_ref[...] = 1.0 / (1.0 + jnp.exp(-t))

    @pl.when(i == 0)
    def _():
        ssum_ref[...] = jnp.zeros_like(ssum_ref)

    ssum_ref[...] += jnp.sum(sig, axis=0, keepdims=True)


def _heads(hrawa, hrawb, gtop, vdirT, wr1az, wvs, wvc, wvi, br1z, wr2p,
           br2p, n):
    bf = 1024
    return pl.pallas_call(
        _head_body,
        grid=(n // bf,),
        in_specs=[
            pl.BlockSpec((bf, 128), lambda i: (i, 0)),
            pl.BlockSpec((bf, 128), lambda i: (i, 0)),
            pl.BlockSpec((bf, 1), lambda i: (i, 0)),
            pl.BlockSpec((3, bf), lambda i: (0, i)),
            pl.BlockSpec((ENC, 256), lambda i: (0, 0)),
            pl.BlockSpec((3 * NDF, 256), lambda i: (0, 0)),
            pl.BlockSpec((3 * NDF, 256), lambda i: (0, 0)),
            pl.BlockSpec((3, 256), lambda i: (0, 0)),
            pl.BlockSpec((1, 256), lambda i: (0, 0)),
            pl.BlockSpec((128, 128), lambda i: (0, 0)),
            pl.BlockSpec((1, 128), lambda i: (0, 0)),
        ],
        out_specs=[
            pl.BlockSpec((bf, 1), lambda i: (i, 0)),
            pl.BlockSpec((bf, 128), lambda i: (i, 0)),
            pl.BlockSpec((1, 1), lambda i: (0, 0)),
        ],
        out_shape=[
            jax.ShapeDtypeStruct((n, 1), F32),
            jax.ShapeDtypeStruct((n, 128), F32),
            jax.ShapeDtypeStruct((1, 1), F32),
        ],
    )(hrawa, hrawb, gtop, vdirT, wr1az, wvs, wvc, wvi,
      br1z, wr2p, br2p)


def kernel(xyz, viewdir, shape_latent, texture_latent, temperature,
           W_enc, b_enc, W_g, b_g,
           We1, be1, We2, be2, We3, be3, We4, be4,
           W_sig, b_sig, W_r1, b_r1, W_r2, b_r2):
    nrays, nsamples, _ = xyz.shape
    n = nrays * nsamples
    npad = (n // T + E) * T

    xyzT = jnp.transpose(xyz, (2, 1, 0)).reshape(3, n)
    vdirT = jnp.transpose(viewdir, (2, 1, 0)).reshape(3, n)
    temp11 = temperature.reshape(1, 1)
    nsf = 3 * NXF
    wsin30 = W_enc[3:3 + nsf]
    wcos30 = W_enc[3 + nsf:3 + 2 * nsf]
    wid3 = W_enc[:3]

    ya, yb, gates, onehot, gtop, counts, gsum = _gating(
        xyzT, temp11, wsin30, wcos30, wid3, b_enc, W_g, b_g, n)

    cnt = counts.reshape(E)
    tile_cnt = jnp.ceil(cnt / T).astype(I32)
    tile_start = jnp.concatenate(
        [jnp.zeros((1,), I32), jnp.cumsum(tile_cnt)[:-1]])
    starts18 = (tile_start * T).astype(F32).reshape(1, E)
    nt = npad // T
    cum = jnp.cumsum(tile_cnt)
    tidx = jnp.arange(nt, dtype=I32)
    tile_eid = jnp.minimum(
        jnp.sum((tidx[:, None] >= cum[None, :]).astype(I32), axis=1),
        E - 1).astype(I32)

    dest = _dest(onehot, starts18, n)
    dest3 = dest.reshape(NWORK, (n // NWORK) // CHUNK, CHUNK)

    ysa, ysb = _dispatch_scatter(ya, yb, dest3, npad)
    hsa, hsb = _experts(tile_eid, ysa, ysb, We1, be1, We2, be2, We3, be3,
                        We4, be4, npad)
    hrawa, hrawb = _combine_gather(hsa, hsb, dest3, n)

    ncf = 3 * NDF
    wr1az = jnp.concatenate(
        [W_r1[:ENC], W_sig, jnp.zeros((ENC, 127), F32)], axis=1)
    wvs = jnp.zeros((ncf, 256), F32).at[:, :128].set(W_r1[ENC + 3:ENC + 3 + ncf])
    wvc = jnp.zeros((ncf, 256), F32).at[:, :128].set(W_r1[ENC + 3 + ncf:])
    wvi = jnp.zeros((3, 256), F32).at[:, :128].set(W_r1[ENC:ENC + 3])
    br1z = jnp.concatenate(
        [b_r1, b_sig, jnp.zeros((127,), F32)]).reshape(1, 256)
    wr2p = jnp.concatenate([W_r2, jnp.zeros((128, 125), F32)], axis=1)
    br2p = jnp.concatenate([b_r2, jnp.zeros((125,), F32)]).reshape(1, 128)

    sig, rgbp, ssum = _heads(hrawa, hrawb, gtop, vdirT, wr1az, wvs, wvc,
                             wvi, br1z, wr2p, br2p, n)

    sigmas = sig.reshape(nsamples, nrays, 1).transpose(1, 0, 2)
    rgbs = rgbp[:, :3].reshape(nsamples, nrays, 3).transpose(1, 0, 2)
    gates_soft_o = gates.reshape(nsamples, nrays, E).transpose(1, 0, 2)
    gates_hard_o = onehot.reshape(nsamples, nrays, E).transpose(1, 0, 2)
    mean_sigma = (ssum / n).reshape(1)
    num_pts = cnt
    aux_loss = E * jnp.sum((cnt / n) * (gsum.reshape(E) / n))
    return (sigmas, rgbs, gates_soft_o, gates_hard_o,
            mean_sigma, num_pts, aux_loss)


# speedup vs baseline: 1.6010x; 1.0497x over previous
import functools

import jax
import jax.numpy as jnp
import numpy as np
from jax import lax
from jax.experimental import pallas as pl
from jax.experimental.pallas import tpu as pltpu
from jax.experimental.pallas import tpu_sc as plsc

F32 = jnp.float32
I32 = jnp.int32

E = 8
ENC = 256
WID = 256
NXF = 10
NDF = 4
T = 2048

SC_CORES = 2
SC_SUBCORES = 16
NWORK = SC_CORES * SC_SUBCORES
CHUNK = 128


BF16 = jnp.bfloat16


def _split3(a):
    ah = a.astype(BF16)
    al = (a - ah.astype(F32)).astype(BF16)
    return ah, al


def _dot3(a, b):
    ah, al = _split3(a)
    bh, bl = _split3(b)
    return (jnp.dot(ah, bh, preferred_element_type=F32)
            + jnp.dot(ah, bl, preferred_element_type=F32)
            + jnp.dot(al, bh, preferred_element_type=F32))


def _dot3_pre(ah, al, bh, bl):
    return (jnp.dot(ah, bh, preferred_element_type=F32)
            + jnp.dot(ah, bl, preferred_element_type=F32)
            + jnp.dot(al, bh, preferred_element_type=F32))


def _pe_matrix(degree, width):
    m = np.zeros((3, width), np.float32)
    half = width // 2
    for d in range(degree):
        for c in range(3):
            m[c, 3 * d + c] = 2.0 ** d
            m[c, half + 3 * d + c] = 2.0 ** d
    return jnp.asarray(m)


def _pe_sincos(x, mat):
    t = (x[:, 0:1] * mat[0:1, :] + x[:, 1:2] * mat[1:2, :]
         + x[:, 2:3] * mat[2:3, :])
    half = t.shape[1] // 2
    return jnp.sin(t[:, :half]), jnp.cos(t[:, half:])


_DN0 = (((0,), (0,)), ((), ()))


def _gating_body(temp_ref, xyzT_ref, wsin_ref, wcos_ref, wid_ref, benc_ref,
                 wg_ref, bg_ref,
                 ya_ref, yb_ref, gates_ref, onehot_ref, gtop_ref,
                 counts_ref, gsum_ref):
    i = pl.program_id(0)
    xt = xyzT_ref[...]
    t30 = jnp.concatenate([xt * (2.0 ** d) for d in range(NXF)], axis=0)
    s = jnp.sin(t30)
    c = jnp.cos(t30)
    y = (lax.dot_general(s, wsin_ref[...], _DN0, preferred_element_type=F32)
         + lax.dot_general(c, wcos_ref[...], _DN0, preferred_element_type=F32)
         + lax.dot_general(xt, wid_ref[...], _DN0, preferred_element_type=F32)
         + benc_ref[...])
    ya_ref[...] = y[:, :128]
    yb_ref[...] = y[:, 128:]
    logits = jnp.dot(y, wg_ref[...], preferred_element_type=F32) + bg_ref[...]
    lt = logits / temp_ref[0, 0]
    m = jnp.max(lt, axis=1, keepdims=True)
    ex = jnp.exp(lt - m)
    g = ex / jnp.sum(ex, axis=1, keepdims=True)
    gates_ref[...] = g
    li = lax.broadcasted_iota(I32, g.shape, 1)
    gm = jnp.max(g, axis=1, keepdims=True)
    am = jnp.min(jnp.where(g == gm, li, E), axis=1, keepdims=True)
    oh = (li == am).astype(F32)
    onehot_ref[...] = oh
    gtop_ref[...] = gm

    @pl.when(i == 0)
    def _():
        counts_ref[...] = jnp.zeros_like(counts_ref)
        gsum_ref[...] = jnp.zeros_like(gsum_ref)

    counts_ref[...] += jnp.sum(oh, axis=0, keepdims=True)
    gsum_ref[...] += jnp.sum(g, axis=0, keepdims=True)


def _gating(xyzT, temp11, wsin30, wcos30, wid3, b_enc, W_g, b_g, n):
    ba = 2048
    grid = (n // ba,)
    return pl.pallas_call(
        _gating_body,
        grid=grid,
        in_specs=[
            pl.BlockSpec(memory_space=pltpu.SMEM),
            pl.BlockSpec((3, ba), lambda i: (0, i)),
            pl.BlockSpec((3 * NXF, ENC), lambda i: (0, 0)),
            pl.BlockSpec((3 * NXF, ENC), lambda i: (0, 0)),
            pl.BlockSpec((3, ENC), lambda i: (0, 0)),
            pl.BlockSpec((1, ENC), lambda i: (0, 0)),
            pl.BlockSpec((ENC, E), lambda i: (0, 0)),
            pl.BlockSpec((1, E), lambda i: (0, 0)),
        ],
        out_specs=[
            pl.BlockSpec((ba, 128), lambda i: (i, 0)),
            pl.BlockSpec((ba, 128), lambda i: (i, 0)),
            pl.BlockSpec((ba, E), lambda i: (i, 0)),
            pl.BlockSpec((ba, E), lambda i: (i, 0)),
            pl.BlockSpec((ba, 1), lambda i: (i, 0)),
            pl.BlockSpec((1, E), lambda i: (0, 0)),
            pl.BlockSpec((1, E), lambda i: (0, 0)),
        ],
        out_shape=[
            jax.ShapeDtypeStruct((n, 128), F32),
            jax.ShapeDtypeStruct((n, 128), F32),
            jax.ShapeDtypeStruct((n, E), F32),
            jax.ShapeDtypeStruct((n, E), F32),
            jax.ShapeDtypeStruct((n, 1), F32),
            jax.ShapeDtypeStruct((1, E), F32),
            jax.ShapeDtypeStruct((1, E), F32),
        ],
    )(temp11, xyzT, wsin30, wcos30, wid3,
      b_enc.reshape(1, ENC), W_g, b_g.reshape(1, E))


def _dest_body(onehot_ref, starts_ref, ltri_ref, dest_ref, carry_ref):
    i = pl.program_id(0)

    @pl.when(i == 0)
    def _():
        carry_ref[...] = jnp.zeros_like(carry_ref)

    oh = onehot_ref[...]
    ranks = jnp.dot(ltri_ref[...], oh, preferred_element_type=F32,
                    precision=lax.Precision.DEFAULT)
    base = starts_ref[...] + carry_ref[...]
    destf = jnp.sum(oh * (base + ranks), axis=1, keepdims=True)
    dest_ref[...] = destf.astype(I32)
    carry_ref[...] += jnp.sum(oh, axis=0, keepdims=True)


def _dest(onehot, starts18, n):
    tb = 512
    r = np.arange(tb)
    ltri = jnp.asarray((r[:, None] > r[None, :]).astype(np.float32))
    return pl.pallas_call(
        _dest_body,
        grid=(n // tb,),
        in_specs=[
            pl.BlockSpec((tb, E), lambda i: (i, 0)),
            pl.BlockSpec((1, E), lambda i: (0, 0)),
            pl.BlockSpec((tb, tb), lambda i: (0, 0)),
        ],
        out_specs=pl.BlockSpec((tb, 1), lambda i: (i, 0)),
        out_shape=jax.ShapeDtypeStruct((n, 1), I32),
        scratch_shapes=[pltpu.VMEM((1, E), F32)],
    )(onehot, starts18, ltri)


def _dispatch_scatter(ya, yb, dest3, npad):
    n = ya.shape[0]
    per_w = n // NWORK
    nchunks = per_w // CHUNK
    mesh = plsc.VectorSubcoreMesh(core_axis_name="c", subcore_axis_name="s")

    @functools.partial(
        pl.kernel,
        mesh=mesh,
        out_type=[jax.ShapeDtypeStruct((npad, 128), F32),
                  jax.ShapeDtypeStruct((npad, 128), F32)],
        scratch_types=[
            pltpu.VMEM((nchunks, CHUNK), I32),
            pltpu.VMEM((CHUNK, 128), F32),
            pltpu.VMEM((CHUNK, 128), F32),
            pltpu.SemaphoreType.DMA,
            pltpu.SemaphoreType.DMA,
        ],
    )
    def k(ya_hbm, yb_hbm, dest_hbm, ysa_hbm, ysb_hbm, idx_v, rowa_v, rowb_v,
          sema, semb):
        wid = lax.axis_index("s") * SC_CORES + lax.axis_index("c")
        pltpu.sync_copy(dest_hbm.at[wid], idx_v)
        base = wid * per_w
        for j in range(nchunks):
            pltpu.sync_copy(ya_hbm.at[pl.ds(base + j * CHUNK, CHUNK)], rowa_v)
            pltpu.sync_copy(yb_hbm.at[pl.ds(base + j * CHUNK, CHUNK)], rowb_v)
            ca = pltpu.async_copy(rowa_v, ysa_hbm.at[idx_v.at[j]], sema)
            cb = pltpu.async_copy(rowb_v, ysb_hbm.at[idx_v.at[j]], semb)
            ca.wait()
            cb.wait()

    return k(ya, yb, dest3)


def _expert_body(eid_ref, ysa_ref, ysb_ref, w1_ref, b1_ref, w2_ref, b2_ref,
                 w3_ref, b3_ref, w4_ref, b4_ref, outa_ref, outb_ref):
    t = pl.program_id(0)
    e = eid_ref[t]
    a = jnp.concatenate([ysa_ref[...], ysb_ref[...]], axis=1)
    h = jnp.maximum(jnp.dot(a, w1_ref[e], preferred_element_type=F32) + b1_ref[e], 0.0)
    h = jnp.maximum(jnp.dot(h, w2_ref[e], preferred_element_type=F32) + b2_ref[e], 0.0)
    h = jnp.maximum(jnp.dot(h, w3_ref[e], preferred_element_type=F32) + b3_ref[e], 0.0)
    h = jnp.dot(h, w4_ref[e], preferred_element_type=F32) + b4_ref[e]
    outa_ref[...] = h[:, :128]
    outb_ref[...] = h[:, 128:]


def _experts(tile_eid, ysa, ysb, We1, be1, We2, be2, We3, be3, We4, be4,
             npad):
    nt = npad // T
    wspec = pl.BlockSpec((E, ENC, WID), lambda t, eid: (0, 0, 0))
    bspec = pl.BlockSpec((E, 1, WID), lambda t, eid: (0, 0, 0))
    grid_spec = pltpu.PrefetchScalarGridSpec(
        num_scalar_prefetch=1,
        grid=(nt,),
        in_specs=[
            pl.BlockSpec((T, 128), lambda t, eid: (t, 0)),
            pl.BlockSpec((T, 128), lambda t, eid: (t, 0)),
            wspec, bspec, wspec, bspec, wspec, bspec, wspec, bspec,
        ],
        out_specs=[pl.BlockSpec((T, 128), lambda t, eid: (t, 0)),
                   pl.BlockSpec((T, 128), lambda t, eid: (t, 0))],
    )
    return pl.pallas_call(
        _expert_body,
        grid_spec=grid_spec,
        out_shape=[jax.ShapeDtypeStruct((npad, 128), F32),
                   jax.ShapeDtypeStruct((npad, 128), F32)],
    )(tile_eid, ysa, ysb,
      We1, be1.reshape(E, 1, WID), We2, be2.reshape(E, 1, WID),
      We3, be3.reshape(E, 1, WID), We4, be4.reshape(E, 1, WID))


def _combine_gather(hsa, hsb, dest3, n):
    per_w = n // NWORK
    nchunks = per_w // CHUNK
    mesh = plsc.VectorSubcoreMesh(core_axis_name="c", subcore_axis_name="s")

    @functools.partial(
        pl.kernel,
        mesh=mesh,
        out_type=[jax.ShapeDtypeStruct((n, 128), F32),
                  jax.ShapeDtypeStruct((n, 128), F32)],
        scratch_types=[
            pltpu.VMEM((nchunks, CHUNK), I32),
            pltpu.VMEM((CHUNK, 128), F32),
            pltpu.VMEM((CHUNK, 128), F32),
            pltpu.SemaphoreType.DMA,
            pltpu.SemaphoreType.DMA,
        ],
    )
    def k(hsa_hbm, hsb_hbm, dest_hbm, outa_hbm, outb_hbm, idx_v, rowa_v,
          rowb_v, sema, semb):
        wid = lax.axis_index("s") * SC_CORES + lax.axis_index("c")
        pltpu.sync_copy(dest_hbm.at[wid], idx_v)
        base = wid * per_w
        for j in range(nchunks):
            ca = pltpu.async_copy(hsa_hbm.at[idx_v.at[j]], rowa_v, sema)
            cb = pltpu.async_copy(hsb_hbm.at[idx_v.at[j]], rowb_v, semb)
            ca.wait()
            cb.wait()
            pltpu.sync_copy(rowa_v, outa_hbm.at[pl.ds(base + j * CHUNK, CHUNK)])
            pltpu.sync_copy(rowb_v, outb_hbm.at[pl.ds(base + j * CHUNK, CHUNK)])

    return k(hsa, hsb, dest3)


def _head_body(hrawa_ref, hrawb_ref, gtop_ref, vdirT_ref,
               wr1az_ref, wvs_ref, wvc_ref, wvi_ref, br1z_ref,
               wr2_ref, br2_ref,
               sig_ref, rgb_ref, ssum_ref):
    i = pl.program_id(0)
    so = jnp.concatenate([hrawa_ref[...], hrawb_ref[...]],
                         axis=1) * gtop_ref[...]
    vt = vdirT_ref[...]
    t12 = jnp.concatenate([vt * (2.0 ** d) for d in range(NDF)], axis=0)
    s = jnp.sin(t12)
    c = jnp.cos(t12)
    u = (jnp.dot(so, wr1az_ref[...], preferred_element_type=F32)
         + lax.dot_general(s, wvs_ref[...], _DN0, preferred_element_type=F32)
         + lax.dot_general(c, wvc_ref[...], _DN0, preferred_element_type=F32)
         + lax.dot_general(vt, wvi_ref[...], _DN0, preferred_element_type=F32)
         + br1z_ref[...])
    z = u[:, 128:129]
    sig = jnp.maximum(z, 0.0) + jnp.log(1.0 + jnp.exp(-jnp.abs(z)))
    sig_ref[...] = sig
    hr = jnp.maximum(u[:, :128], 0.0)
    t = jnp.dot(hr, wr2_ref[...], preferred_element_type=F32) + br2_ref[...]
    rgb_ref[...] = 1.0 / (1.0 + jnp.exp(-t))

    @pl.when(i == 0)
    def _():
        ssum_ref[...] = jnp.zeros_like(ssum_ref)

    ssum_ref[...] += jnp.sum(sig, axis=0, keepdims=True)


def _heads(hrawa, hrawb, gtop, vdirT, wr1az, wvs, wvc, wvi, br1z, wr2p,
           br2p, n):
    bf = 2048
    return pl.pallas_call(
        _head_body,
        grid=(n // bf,),
        in_specs=[
            pl.BlockSpec((bf, 128), lambda i: (i, 0)),
            pl.BlockSpec((bf, 128), lambda i: (i, 0)),
            pl.BlockSpec((bf, 1), lambda i: (i, 0)),
            pl.BlockSpec((3, bf), lambda i: (0, i)),
            pl.BlockSpec((ENC, 256), lambda i: (0, 0)),
            pl.BlockSpec((3 * NDF, 256), lambda i: (0, 0)),
            pl.BlockSpec((3 * NDF, 256), lambda i: (0, 0)),
            pl.BlockSpec((3, 256), lambda i: (0, 0)),
            pl.BlockSpec((1, 256), lambda i: (0, 0)),
            pl.BlockSpec((128, 128), lambda i: (0, 0)),
            pl.BlockSpec((1, 128), lambda i: (0, 0)),
        ],
        out_specs=[
            pl.BlockSpec((bf, 1), lambda i: (i, 0)),
            pl.BlockSpec((bf, 128), lambda i: (i, 0)),
            pl.BlockSpec((1, 1), lambda i: (0, 0)),
        ],
        out_shape=[
            jax.ShapeDtypeStruct((n, 1), F32),
            jax.ShapeDtypeStruct((n, 128), F32),
            jax.ShapeDtypeStruct((1, 1), F32),
        ],
    )(hrawa, hrawb, gtop, vdirT, wr1az, wvs, wvc, wvi,
      br1z, wr2p, br2p)


def kernel(xyz, viewdir, shape_latent, texture_latent, temperature,
           W_enc, b_enc, W_g, b_g,
           We1, be1, We2, be2, We3, be3, We4, be4,
           W_sig, b_sig, W_r1, b_r1, W_r2, b_r2):
    nrays, nsamples, _ = xyz.shape
    n = nrays * nsamples
    npad = (n // T + E) * T

    xyzT = jnp.transpose(xyz, (2, 1, 0)).reshape(3, n)
    vdirT = jnp.transpose(viewdir, (2, 1, 0)).reshape(3, n)
    temp11 = temperature.reshape(1, 1)
    nsf = 3 * NXF
    wsin30 = W_enc[3:3 + nsf]
    wcos30 = W_enc[3 + nsf:3 + 2 * nsf]
    wid3 = W_enc[:3]

    ya, yb, gates, onehot, gtop, counts, gsum = _gating(
        xyzT, temp11, wsin30, wcos30, wid3, b_enc, W_g, b_g, n)

    cnt = counts.reshape(E)
    tile_cnt = jnp.ceil(cnt / T).astype(I32)
    tile_start = jnp.concatenate(
        [jnp.zeros((1,), I32), jnp.cumsum(tile_cnt)[:-1]])
    starts18 = (tile_start * T).astype(F32).reshape(1, E)
    nt = npad // T
    cum = jnp.cumsum(tile_cnt)
    tidx = jnp.arange(nt, dtype=I32)
    tile_eid = jnp.minimum(
        jnp.sum((tidx[:, None] >= cum[None, :]).astype(I32), axis=1),
        E - 1).astype(I32)

    dest = _dest(onehot, starts18, n)
    dest3 = dest.reshape(NWORK, (n // NWORK) // CHUNK, CHUNK)

    ysa, ysb = _dispatch_scatter(ya, yb, dest3, npad)
    hsa, hsb = _experts(tile_eid, ysa, ysb, We1, be1, We2, be2, We3, be3,
                        We4, be4, npad)
    hrawa, hrawb = _combine_gather(hsa, hsb, dest3, n)

    ncf = 3 * NDF
    wr1az = jnp.concatenate(
        [W_r1[:ENC], W_sig, jnp.zeros((ENC, 127), F32)], axis=1)
    wvs = jnp.zeros((ncf, 256), F32).at[:, :128].set(W_r1[ENC + 3:ENC + 3 + ncf])
    wvc = jnp.zeros((ncf, 256), F32).at[:, :128].set(W_r1[ENC + 3 + ncf:])
    wvi = jnp.zeros((3, 256), F32).at[:, :128].set(W_r1[ENC:ENC + 3])
    br1z = jnp.concatenate(
        [b_r1, b_sig, jnp.zeros((127,), F32)]).reshape(1, 256)
    wr2p = jnp.concatenate([W_r2, jnp.zeros((128, 125), F32)], axis=1)
    br2p = jnp.concatenate([b_r2, jnp.zeros((125,), F32)]).reshape(1, 128)

    sig, rgbp, ssum = _heads(hrawa, hrawb, gtop, vdirT, wr1az, wvs, wvc,
                             wvi, br1z, wr2p, br2p, n)

    sigmas = sig.reshape(nsamples, nrays, 1).transpose(1, 0, 2)
    rgbs = rgbp[:, :3].reshape(nsamples, nrays, 3).transpose(1, 0, 2)
    gates_soft_o = gates.reshape(nsamples, nrays, E).transpose(1, 0, 2)
    gates_hard_o = onehot.reshape(nsamples, nrays, E).transpose(1, 0, 2)
    mean_sigma = (ssum / n).reshape(1)
    num_pts = cnt
    aux_loss = E * jnp.sum((cnt / n) * (gsum.reshape(E) / n))
    return (sigmas, rgbs, gates_soft_o, gates_hard_o,
            mean_sigma, num_pts, aux_loss)


# speedup vs baseline: 1.6342x; 1.0207x over previous
import functools

import jax
import jax.numpy as jnp
import numpy as np
from jax import lax
from jax.experimental import pallas as pl
from jax.experimental.pallas import tpu as pltpu
from jax.experimental.pallas import tpu_sc as plsc

F32 = jnp.float32
I32 = jnp.int32

E = 8
ENC = 256
WID = 256
NXF = 10
NDF = 4
T = 2048

SC_CORES = 2
SC_SUBCORES = 16
NWORK = SC_CORES * SC_SUBCORES
CHUNK = 128


BF16 = jnp.bfloat16


def _split3(a):
    ah = a.astype(BF16)
    al = (a - ah.astype(F32)).astype(BF16)
    return ah, al


def _dot3(a, b):
    ah, al = _split3(a)
    bh, bl = _split3(b)
    return (jnp.dot(ah, bh, preferred_element_type=F32)
            + jnp.dot(ah, bl, preferred_element_type=F32)
            + jnp.dot(al, bh, preferred_element_type=F32))


def _dot3_pre(ah, al, bh, bl):
    return (jnp.dot(ah, bh, preferred_element_type=F32)
            + jnp.dot(ah, bl, preferred_element_type=F32)
            + jnp.dot(al, bh, preferred_element_type=F32))


def _pe_matrix(degree, width):
    m = np.zeros((3, width), np.float32)
    half = width // 2
    for d in range(degree):
        for c in range(3):
            m[c, 3 * d + c] = 2.0 ** d
            m[c, half + 3 * d + c] = 2.0 ** d
    return jnp.asarray(m)


def _pe_sincos(x, mat):
    t = (x[:, 0:1] * mat[0:1, :] + x[:, 1:2] * mat[1:2, :]
         + x[:, 2:3] * mat[2:3, :])
    half = t.shape[1] // 2
    return jnp.sin(t[:, :half]), jnp.cos(t[:, half:])


_DN0 = (((0,), (0,)), ((), ()))


def _gating_body(temp_ref, xyzT_ref, wsin_ref, wcos_ref, wid_ref, benc_ref,
                 wg_ref, bg_ref,
                 ya_ref, yb_ref, gates_ref, onehot_ref, gtop_ref,
                 counts_ref, gsum_ref):
    i = pl.program_id(0)
    xt = xyzT_ref[...]
    t30 = jnp.concatenate([xt * (2.0 ** d) for d in range(NXF)], axis=0)
    s = jnp.sin(t30)
    c = jnp.cos(t30)
    y = (lax.dot_general(s, wsin_ref[...], _DN0, preferred_element_type=F32)
         + lax.dot_general(c, wcos_ref[...], _DN0, preferred_element_type=F32)
         + lax.dot_general(xt, wid_ref[...], _DN0, preferred_element_type=F32)
         + benc_ref[...])
    ya_ref[...] = y[:, :128]
    yb_ref[...] = y[:, 128:]
    logits = jnp.dot(y, wg_ref[...], preferred_element_type=F32) + bg_ref[...]
    lt = logits / temp_ref[0, 0]
    m = jnp.max(lt, axis=1, keepdims=True)
    ex = jnp.exp(lt - m)
    g = ex / jnp.sum(ex, axis=1, keepdims=True)
    gates_ref[...] = g
    li = lax.broadcasted_iota(I32, g.shape, 1)
    gm = jnp.max(g, axis=1, keepdims=True)
    am = jnp.min(jnp.where(g == gm, li, E), axis=1, keepdims=True)
    oh = (li == am).astype(F32)
    onehot_ref[...] = oh
    gtop_ref[...] = gm

    @pl.when(i == 0)
    def _():
        counts_ref[...] = jnp.zeros_like(counts_ref)
        gsum_ref[...] = jnp.zeros_like(gsum_ref)

    counts_ref[...] += jnp.sum(oh, axis=0, keepdims=True)
    gsum_ref[...] += jnp.sum(g, axis=0, keepdims=True)


def _gating(xyzT, temp11, wsin30, wcos30, wid3, b_enc, W_g, b_g, n):
    ba = 4096
    grid = (n // ba,)
    return pl.pallas_call(
        _gating_body,
        grid=grid,
        in_specs=[
            pl.BlockSpec(memory_space=pltpu.SMEM),
            pl.BlockSpec((3, ba), lambda i: (0, i)),
            pl.BlockSpec((3 * NXF, ENC), lambda i: (0, 0)),
            pl.BlockSpec((3 * NXF, ENC), lambda i: (0, 0)),
            pl.BlockSpec((3, ENC), lambda i: (0, 0)),
            pl.BlockSpec((1, ENC), lambda i: (0, 0)),
            pl.BlockSpec((ENC, E), lambda i: (0, 0)),
            pl.BlockSpec((1, E), lambda i: (0, 0)),
        ],
        out_specs=[
            pl.BlockSpec((ba, 128), lambda i: (i, 0)),
            pl.BlockSpec((ba, 128), lambda i: (i, 0)),
            pl.BlockSpec((ba, E), lambda i: (i, 0)),
            pl.BlockSpec((ba, E), lambda i: (i, 0)),
            pl.BlockSpec((ba, 1), lambda i: (i, 0)),
            pl.BlockSpec((1, E), lambda i: (0, 0)),
            pl.BlockSpec((1, E), lambda i: (0, 0)),
        ],
        out_shape=[
            jax.ShapeDtypeStruct((n, 128), F32),
            jax.ShapeDtypeStruct((n, 128), F32),
            jax.ShapeDtypeStruct((n, E), F32),
            jax.ShapeDtypeStruct((n, E), F32),
            jax.ShapeDtypeStruct((n, 1), F32),
            jax.ShapeDtypeStruct((1, E), F32),
            jax.ShapeDtypeStruct((1, E), F32),
        ],
    )(temp11, xyzT, wsin30, wcos30, wid3,
      b_enc.reshape(1, ENC), W_g, b_g.reshape(1, E))


def _dest_body(onehot_ref, starts_ref, ltri_ref, dest_ref, carry_ref):
    i = pl.program_id(0)

    @pl.when(i == 0)
    def _():
        carry_ref[...] = jnp.zeros_like(carry_ref)

    oh = onehot_ref[...]
    ranks = jnp.dot(ltri_ref[...], oh, preferred_element_type=F32,
                    precision=lax.Precision.DEFAULT)
    base = starts_ref[...] + carry_ref[...]
    destf = jnp.sum(oh * (base + ranks), axis=1, keepdims=True)
    dest_ref[...] = destf.astype(I32)
    carry_ref[...] += jnp.sum(oh, axis=0, keepdims=True)


def _dest(onehot, starts18, n):
    tb = 512
    r = np.arange(tb)
    ltri = jnp.asarray((r[:, None] > r[None, :]).astype(np.float32))
    return pl.pallas_call(
        _dest_body,
        grid=(n // tb,),
        in_specs=[
            pl.BlockSpec((tb, E), lambda i: (i, 0)),
            pl.BlockSpec((1, E), lambda i: (0, 0)),
            pl.BlockSpec((tb, tb), lambda i: (0, 0)),
        ],
        out_specs=pl.BlockSpec((tb, 1), lambda i: (i, 0)),
        out_shape=jax.ShapeDtypeStruct((n, 1), I32),
        scratch_shapes=[pltpu.VMEM((1, E), F32)],
    )(onehot, starts18, ltri)


def _dispatch_scatter(ya, yb, dest3, npad):
    n = ya.shape[0]
    per_w = n // NWORK
    nchunks = per_w // CHUNK
    mesh = plsc.VectorSubcoreMesh(core_axis_name="c", subcore_axis_name="s")

    @functools.partial(
        pl.kernel,
        mesh=mesh,
        out_type=[jax.ShapeDtypeStruct((npad, 128), F32),
                  jax.ShapeDtypeStruct((npad, 128), F32)],
        scratch_types=[
            pltpu.VMEM((nchunks, CHUNK), I32),
            pltpu.VMEM((CHUNK, 128), F32),
            pltpu.VMEM((CHUNK, 128), F32),
            pltpu.SemaphoreType.DMA,
            pltpu.SemaphoreType.DMA,
        ],
    )
    def k(ya_hbm, yb_hbm, dest_hbm, ysa_hbm, ysb_hbm, idx_v, rowa_v, rowb_v,
          sema, semb):
        wid = lax.axis_index("s") * SC_CORES + lax.axis_index("c")
        pltpu.sync_copy(dest_hbm.at[wid], idx_v)
        base = wid * per_w
        for j in range(nchunks):
            pltpu.sync_copy(ya_hbm.at[pl.ds(base + j * CHUNK, CHUNK)], rowa_v)
            pltpu.sync_copy(yb_hbm.at[pl.ds(base + j * CHUNK, CHUNK)], rowb_v)
            ca = pltpu.async_copy(rowa_v, ysa_hbm.at[idx_v.at[j]], sema)
            cb = pltpu.async_copy(rowb_v, ysb_hbm.at[idx_v.at[j]], semb)
            ca.wait()
            cb.wait()

    return k(ya, yb, dest3)


def _expert_body(eid_ref, ysa_ref, ysb_ref, w1_ref, b1_ref, w2_ref, b2_ref,
                 w3_ref, b3_ref, w4_ref, b4_ref, outa_ref, outb_ref):
    t = pl.program_id(0)
    e = eid_ref[t]
    a = jnp.concatenate([ysa_ref[...], ysb_ref[...]], axis=1)
    h = jnp.maximum(jnp.dot(a, w1_ref[e], preferred_element_type=F32) + b1_ref[e], 0.0)
    h = jnp.maximum(jnp.dot(h, w2_ref[e], preferred_element_type=F32) + b2_ref[e], 0.0)
    h = jnp.maximum(jnp.dot(h, w3_ref[e], preferred_element_type=F32) + b3_ref[e], 0.0)
    h = jnp.dot(h, w4_ref[e], preferred_element_type=F32) + b4_ref[e]
    outa_ref[...] = h[:, :128]
    outb_ref[...] = h[:, 128:]


def _experts(tile_eid, ysa, ysb, We1, be1, We2, be2, We3, be3, We4, be4,
             npad):
    nt = npad // T
    wspec = pl.BlockSpec((E, ENC, WID), lambda t, eid: (0, 0, 0))
    bspec = pl.BlockSpec((E, 1, WID), lambda t, eid: (0, 0, 0))
    grid_spec = pltpu.PrefetchScalarGridSpec(
        num_scalar_prefetch=1,
        grid=(nt,),
        in_specs=[
            pl.BlockSpec((T, 128), lambda t, eid: (t, 0)),
            pl.BlockSpec((T, 128), lambda t, eid: (t, 0)),
            wspec, bspec, wspec, bspec, wspec, bspec, wspec, bspec,
        ],
        out_specs=[pl.BlockSpec((T, 128), lambda t, eid: (t, 0)),
                   pl.BlockSpec((T, 128), lambda t, eid: (t, 0))],
    )
    return pl.pallas_call(
        _expert_body,
        grid_spec=grid_spec,
        out_shape=[jax.ShapeDtypeStruct((npad, 128), F32),
                   jax.ShapeDtypeStruct((npad, 128), F32)],
    )(tile_eid, ysa, ysb,
      We1, be1.reshape(E, 1, WID), We2, be2.reshape(E, 1, WID),
      We3, be3.reshape(E, 1, WID), We4, be4.reshape(E, 1, WID))


def _combine_gather(hsa, hsb, dest3, n):
    per_w = n // NWORK
    nchunks = per_w // CHUNK
    mesh = plsc.VectorSubcoreMesh(core_axis_name="c", subcore_axis_name="s")

    @functools.partial(
        pl.kernel,
        mesh=mesh,
        out_type=[jax.ShapeDtypeStruct((n, 128), F32),
                  jax.ShapeDtypeStruct((n, 128), F32)],
        scratch_types=[
            pltpu.VMEM((nchunks, CHUNK), I32),
            pltpu.VMEM((CHUNK, 128), F32),
            pltpu.VMEM((CHUNK, 128), F32),
            pltpu.SemaphoreType.DMA,
            pltpu.SemaphoreType.DMA,
        ],
    )
    def k(hsa_hbm, hsb_hbm, dest_hbm, outa_hbm, outb_hbm, idx_v, rowa_v,
          rowb_v, sema, semb):
        wid = lax.axis_index("s") * SC_CORES + lax.axis_index("c")
        pltpu.sync_copy(dest_hbm.at[wid], idx_v)
        base = wid * per_w
        for j in range(nchunks):
            ca = pltpu.async_copy(hsa_hbm.at[idx_v.at[j]], rowa_v, sema)
            cb = pltpu.async_copy(hsb_hbm.at[idx_v.at[j]], rowb_v, semb)
            ca.wait()
            cb.wait()
            pltpu.sync_copy(rowa_v, outa_hbm.at[pl.ds(base + j * CHUNK, CHUNK)])
            pltpu.sync_copy(rowb_v, outb_hbm.at[pl.ds(base + j * CHUNK, CHUNK)])

    return k(hsa, hsb, dest3)


def _head_body(hrawa_ref, hrawb_ref, gtop_ref, vdirT_ref,
               wr1az_ref, wvs_ref, wvc_ref, wvi_ref, br1z_ref,
               wr2_ref, br2_ref,
               sig_ref, rgb_ref, ssum_ref):
    i = pl.program_id(0)
    so = jnp.concatenate([hrawa_ref[...], hrawb_ref[...]],
                         axis=1) * gtop_ref[...]
    vt = vdirT_ref[...]
    t12 = jnp.concatenate([vt * (2.0 ** d) for d in range(NDF)], axis=0)
    s = jnp.sin(t12)
    c = jnp.cos(t12)
    u = (jnp.dot(so, wr1az_ref[...], preferred_element_type=F32)
         + lax.dot_general(s, wvs_ref[...], _DN0, preferred_element_type=F32)
         + lax.dot_general(c, wvc_ref[...], _DN0, preferred_element_type=F32)
         + lax.dot_general(vt, wvi_ref[...], _DN0, preferred_element_type=F32)
         + br1z_ref[...])
    z = u[:, 128:129]
    sig = jnp.maximum(z, 0.0) + jnp.log(1.0 + jnp.exp(-jnp.abs(z)))
    sig_ref[...] = sig
    hr = jnp.maximum(u[:, :128], 0.0)
    t = jnp.dot(hr, wr2_ref[...], preferred_element_type=F32) + br2_ref[...]
    rgb_ref[...] = 1.0 / (1.0 + jnp.exp(-t))

    @pl.when(i == 0)
    def _():
        ssum_ref[...] = jnp.zeros_like(ssum_ref)

    ssum_ref[...] += jnp.sum(sig, axis=0, keepdims=True)


def _heads(hrawa, hrawb, gtop, vdirT, wr1az, wvs, wvc, wvi, br1z, wr2p,
           br2p, n):
    bf = 4096
    return pl.pallas_call(
        _head_body,
        grid=(n // bf,),
        in_specs=[
            pl.BlockSpec((bf, 128), lambda i: (i, 0)),
            pl.BlockSpec((bf, 128), lambda i: (i, 0)),
            pl.BlockSpec((bf, 1), lambda i: (i, 0)),
            pl.BlockSpec((3, bf), lambda i: (0, i)),
            pl.BlockSpec((ENC, 256), lambda i: (0, 0)),
            pl.BlockSpec((3 * NDF, 256), lambda i: (0, 0)),
            pl.BlockSpec((3 * NDF, 256), lambda i: (0, 0)),
            pl.BlockSpec((3, 256), lambda i: (0, 0)),
            pl.BlockSpec((1, 256), lambda i: (0, 0)),
            pl.BlockSpec((128, 128), lambda i: (0, 0)),
            pl.BlockSpec((1, 128), lambda i: (0, 0)),
        ],
        out_specs=[
            pl.BlockSpec((bf, 1), lambda i: (i, 0)),
            pl.BlockSpec((bf, 128), lambda i: (i, 0)),
            pl.BlockSpec((1, 1), lambda i: (0, 0)),
        ],
        out_shape=[
            jax.ShapeDtypeStruct((n, 1), F32),
            jax.ShapeDtypeStruct((n, 128), F32),
            jax.ShapeDtypeStruct((1, 1), F32),
        ],
    )(hrawa, hrawb, gtop, vdirT, wr1az, wvs, wvc, wvi,
      br1z, wr2p, br2p)


def kernel(xyz, viewdir, shape_latent, texture_latent, temperature,
           W_enc, b_enc, W_g, b_g,
           We1, be1, We2, be2, We3, be3, We4, be4,
           W_sig, b_sig, W_r1, b_r1, W_r2, b_r2):
    nrays, nsamples, _ = xyz.shape
    n = nrays * nsamples
    npad = (n // T + E) * T

    xyzT = jnp.transpose(xyz, (2, 1, 0)).reshape(3, n)
    vdirT = jnp.transpose(viewdir, (2, 1, 0)).reshape(3, n)
    temp11 = temperature.reshape(1, 1)
    nsf = 3 * NXF
    wsin30 = W_enc[3:3 + nsf]
    wcos30 = W_enc[3 + nsf:3 + 2 * nsf]
    wid3 = W_enc[:3]

    ya, yb, gates, onehot, gtop, counts, gsum = _gating(
        xyzT, temp11, wsin30, wcos30, wid3, b_enc, W_g, b_g, n)

    cnt = counts.reshape(E)
    tile_cnt = jnp.ceil(cnt / T).astype(I32)
    tile_start = jnp.concatenate(
        [jnp.zeros((1,), I32), jnp.cumsum(tile_cnt)[:-1]])
    starts18 = (tile_start * T).astype(F32).reshape(1, E)
    nt = npad // T
    cum = jnp.cumsum(tile_cnt)
    tidx = jnp.arange(nt, dtype=I32)
    tile_eid = jnp.minimum(
        jnp.sum((tidx[:, None] >= cum[None, :]).astype(I32), axis=1),
        E - 1).astype(I32)

    dest = _dest(onehot, starts18, n)
    dest3 = dest.reshape(NWORK, (n // NWORK) // CHUNK, CHUNK)

    ysa, ysb = _dispatch_scatter(ya, yb, dest3, npad)
    hsa, hsb = _experts(tile_eid, ysa, ysb, We1, be1, We2, be2, We3, be3,
                        We4, be4, npad)
    hrawa, hrawb = _combine_gather(hsa, hsb, dest3, n)

    ncf = 3 * NDF
    wr1az = jnp.concatenate(
        [W_r1[:ENC], W_sig, jnp.zeros((ENC, 127), F32)], axis=1)
    wvs = jnp.zeros((ncf, 256), F32).at[:, :128].set(W_r1[ENC + 3:ENC + 3 + ncf])
    wvc = jnp.zeros((ncf, 256), F32).at[:, :128].set(W_r1[ENC + 3 + ncf:])
    wvi = jnp.zeros((3, 256), F32).at[:, :128].set(W_r1[ENC:ENC + 3])
    br1z = jnp.concatenate(
        [b_r1, b_sig, jnp.zeros((127,), F32)]).reshape(1, 256)
    wr2p = jnp.concatenate([W_r2, jnp.zeros((128, 125), F32)], axis=1)
    br2p = jnp.concatenate([b_r2, jnp.zeros((125,), F32)]).reshape(1, 128)

    sig, rgbp, ssum = _heads(hrawa, hrawb, gtop, vdirT, wr1az, wvs, wvc,
                             wvi, br1z, wr2p, br2p, n)

    sigmas = sig.reshape(nsamples, nrays, 1).transpose(1, 0, 2)
    rgbs = rgbp[:, :3].reshape(nsamples, nrays, 3).transpose(1, 0, 2)
    gates_soft_o = gates.reshape(nsamples, nrays, E).transpose(1, 0, 2)
    gates_hard_o = onehot.reshape(nsamples, nrays, E).transpose(1, 0, 2)
    mean_sigma = (ssum / n).reshape(1)
    num_pts = cnt
    aux_loss = E * jnp.sum((cnt / n) * (gsum.reshape(E) / n))
    return (sigmas, rgbs, gates_soft_o, gates_hard_o,
            mean_sigma, num_pts, aux_loss)


# speedup vs baseline: 1.7012x; 1.0410x over previous
import functools

import jax
import jax.numpy as jnp
import numpy as np
from jax import lax
from jax.experimental import pallas as pl
from jax.experimental.pallas import tpu as pltpu
from jax.experimental.pallas import tpu_sc as plsc

F32 = jnp.float32
I32 = jnp.int32

E = 8
ENC = 256
WID = 256
NXF = 10
NDF = 4
T = 2048

SC_CORES = 2
SC_SUBCORES = 16
NWORK = SC_CORES * SC_SUBCORES
CHUNK = 128


BF16 = jnp.bfloat16


def _split3(a):
    ah = a.astype(BF16)
    al = (a - ah.astype(F32)).astype(BF16)
    return ah, al


def _dot3(a, b):
    ah, al = _split3(a)
    bh, bl = _split3(b)
    return (jnp.dot(ah, bh, preferred_element_type=F32)
            + jnp.dot(ah, bl, preferred_element_type=F32)
            + jnp.dot(al, bh, preferred_element_type=F32))


def _dot3_pre(ah, al, bh, bl):
    return (jnp.dot(ah, bh, preferred_element_type=F32)
            + jnp.dot(ah, bl, preferred_element_type=F32)
            + jnp.dot(al, bh, preferred_element_type=F32))


def _pe_matrix(degree, width):
    m = np.zeros((3, width), np.float32)
    half = width // 2
    for d in range(degree):
        for c in range(3):
            m[c, 3 * d + c] = 2.0 ** d
            m[c, half + 3 * d + c] = 2.0 ** d
    return jnp.asarray(m)


def _pe_sincos(x, mat):
    t = (x[:, 0:1] * mat[0:1, :] + x[:, 1:2] * mat[1:2, :]
         + x[:, 2:3] * mat[2:3, :])
    half = t.shape[1] // 2
    return jnp.sin(t[:, :half]), jnp.cos(t[:, half:])


_DN0 = (((0,), (0,)), ((), ()))


def _gating_body(temp_ref, xyzT_ref, wsin_ref, wcos_ref, wid_ref, benc_ref,
                 wg_ref, bg_ref,
                 ya_ref, yb_ref, gates_ref, onehot_ref, gtop_ref,
                 counts_ref, gsum_ref):
    i = pl.program_id(0)
    xt = xyzT_ref[...]
    t30 = jnp.concatenate([xt * (2.0 ** d) for d in range(NXF)], axis=0)
    s = jnp.sin(t30)
    c = jnp.cos(t30)
    y = (lax.dot_general(s, wsin_ref[...], _DN0, preferred_element_type=F32)
         + lax.dot_general(c, wcos_ref[...], _DN0, preferred_element_type=F32)
         + lax.dot_general(xt, wid_ref[...], _DN0, preferred_element_type=F32)
         + benc_ref[...])
    ya_ref[...] = y[:, :128]
    yb_ref[...] = y[:, 128:]
    logits = jnp.dot(y, wg_ref[...], preferred_element_type=F32) + bg_ref[...]
    lt = logits / temp_ref[0, 0]
    m = jnp.max(lt, axis=1, keepdims=True)
    ex = jnp.exp(lt - m)
    g = ex / jnp.sum(ex, axis=1, keepdims=True)
    gates_ref[...] = g
    li = lax.broadcasted_iota(I32, g.shape, 1)
    gm = jnp.max(g, axis=1, keepdims=True)
    am = jnp.min(jnp.where(g == gm, li, E), axis=1, keepdims=True)
    oh = (li == am).astype(F32)
    onehot_ref[...] = oh
    gtop_ref[...] = gm

    @pl.when(i == 0)
    def _():
        counts_ref[...] = jnp.zeros_like(counts_ref)
        gsum_ref[...] = jnp.zeros_like(gsum_ref)

    counts_ref[...] += jnp.sum(oh, axis=0, keepdims=True)
    gsum_ref[...] += jnp.sum(g, axis=0, keepdims=True)


def _gating(xyzT, temp11, wsin30, wcos30, wid3, b_enc, W_g, b_g, n):
    ba = 4096
    grid = (n // ba,)
    return pl.pallas_call(
        _gating_body,
        grid=grid,
        in_specs=[
            pl.BlockSpec(memory_space=pltpu.SMEM),
            pl.BlockSpec((3, ba), lambda i: (0, i)),
            pl.BlockSpec((3 * NXF, ENC), lambda i: (0, 0)),
            pl.BlockSpec((3 * NXF, ENC), lambda i: (0, 0)),
            pl.BlockSpec((3, ENC), lambda i: (0, 0)),
            pl.BlockSpec((1, ENC), lambda i: (0, 0)),
            pl.BlockSpec((ENC, E), lambda i: (0, 0)),
            pl.BlockSpec((1, E), lambda i: (0, 0)),
        ],
        out_specs=[
            pl.BlockSpec((ba, 128), lambda i: (i, 0)),
            pl.BlockSpec((ba, 128), lambda i: (i, 0)),
            pl.BlockSpec((ba, E), lambda i: (i, 0)),
            pl.BlockSpec((ba, E), lambda i: (i, 0)),
            pl.BlockSpec((ba, 1), lambda i: (i, 0)),
            pl.BlockSpec((1, E), lambda i: (0, 0)),
            pl.BlockSpec((1, E), lambda i: (0, 0)),
        ],
        out_shape=[
            jax.ShapeDtypeStruct((n, 128), F32),
            jax.ShapeDtypeStruct((n, 128), F32),
            jax.ShapeDtypeStruct((n, E), F32),
            jax.ShapeDtypeStruct((n, E), F32),
            jax.ShapeDtypeStruct((n, 1), F32),
            jax.ShapeDtypeStruct((1, E), F32),
            jax.ShapeDtypeStruct((1, E), F32),
        ],
    )(temp11, xyzT, wsin30, wcos30, wid3,
      b_enc.reshape(1, ENC), W_g, b_g.reshape(1, E))


def _dest_body(onehot_ref, starts_ref, ltri_ref, dest_ref, carry_ref):
    i = pl.program_id(0)

    @pl.when(i == 0)
    def _():
        carry_ref[...] = jnp.zeros_like(carry_ref)

    oh = onehot_ref[...]
    ranks = jnp.dot(ltri_ref[...], oh, preferred_element_type=F32,
                    precision=lax.Precision.DEFAULT)
    base = starts_ref[...] + carry_ref[...]
    destf = jnp.sum(oh * (base + ranks), axis=1, keepdims=True)
    dest_ref[...] = destf.astype(I32)
    carry_ref[...] += jnp.sum(oh, axis=0, keepdims=True)


def _dest(onehot, starts18, n):
    tb = 1024
    r = np.arange(tb)
    ltri = jnp.asarray((r[:, None] > r[None, :]).astype(np.float32))
    return pl.pallas_call(
        _dest_body,
        grid=(n // tb,),
        in_specs=[
            pl.BlockSpec((tb, E), lambda i: (i, 0)),
            pl.BlockSpec((1, E), lambda i: (0, 0)),
            pl.BlockSpec((tb, tb), lambda i: (0, 0)),
        ],
        out_specs=pl.BlockSpec((tb, 1), lambda i: (i, 0)),
        out_shape=jax.ShapeDtypeStruct((n, 1), I32),
        scratch_shapes=[pltpu.VMEM((1, E), F32)],
    )(onehot, starts18, ltri)


def _dispatch_scatter(ya, yb, dest3, npad):
    n = ya.shape[0]
    per_w = n // NWORK
    nchunks = per_w // CHUNK
    mesh = plsc.VectorSubcoreMesh(core_axis_name="c", subcore_axis_name="s")

    @functools.partial(
        pl.kernel,
        mesh=mesh,
        out_type=[jax.ShapeDtypeStruct((npad, 128), F32),
                  jax.ShapeDtypeStruct((npad, 128), F32)],
        scratch_types=[
            pltpu.VMEM((nchunks, CHUNK), I32),
            pltpu.VMEM((CHUNK, 128), F32),
            pltpu.VMEM((CHUNK, 128), F32),
            pltpu.SemaphoreType.DMA,
            pltpu.SemaphoreType.DMA,
        ],
    )
    def k(ya_hbm, yb_hbm, dest_hbm, ysa_hbm, ysb_hbm, idx_v, rowa_v, rowb_v,
          sema, semb):
        wid = lax.axis_index("s") * SC_CORES + lax.axis_index("c")
        pltpu.sync_copy(dest_hbm.at[wid], idx_v)
        base = wid * per_w
        for j in range(nchunks):
            pltpu.sync_copy(ya_hbm.at[pl.ds(base + j * CHUNK, CHUNK)], rowa_v)
            pltpu.sync_copy(yb_hbm.at[pl.ds(base + j * CHUNK, CHUNK)], rowb_v)
            ca = pltpu.async_copy(rowa_v, ysa_hbm.at[idx_v.at[j]], sema)
            cb = pltpu.async_copy(rowb_v, ysb_hbm.at[idx_v.at[j]], semb)
            ca.wait()
            cb.wait()

    return k(ya, yb, dest3)


def _expert_body(eid_ref, ysa_ref, ysb_ref, w1_ref, b1_ref, w2_ref, b2_ref,
                 w3_ref, b3_ref, w4_ref, b4_ref, outa_ref, outb_ref):
    t = pl.program_id(0)
    e = eid_ref[t]
    a = jnp.concatenate([ysa_ref[...], ysb_ref[...]], axis=1)
    h = jnp.maximum(jnp.dot(a, w1_ref[e], preferred_element_type=F32) + b1_ref[e], 0.0)
    h = jnp.maximum(jnp.dot(h, w2_ref[e], preferred_element_type=F32) + b2_ref[e], 0.0)
    h = jnp.maximum(jnp.dot(h, w3_ref[e], preferred_element_type=F32) + b3_ref[e], 0.0)
    h = jnp.dot(h, w4_ref[e], preferred_element_type=F32) + b4_ref[e]
    outa_ref[...] = h[:, :128]
    outb_ref[...] = h[:, 128:]


def _experts(tile_eid, ysa, ysb, We1, be1, We2, be2, We3, be3, We4, be4,
             npad):
    nt = npad // T
    wspec = pl.BlockSpec((E, ENC, WID), lambda t, eid: (0, 0, 0))
    bspec = pl.BlockSpec((E, 1, WID), lambda t, eid: (0, 0, 0))
    grid_spec = pltpu.PrefetchScalarGridSpec(
        num_scalar_prefetch=1,
        grid=(nt,),
        in_specs=[
            pl.BlockSpec((T, 128), lambda t, eid: (t, 0)),
            pl.BlockSpec((T, 128), lambda t, eid: (t, 0)),
            wspec, bspec, wspec, bspec, wspec, bspec, wspec, bspec,
        ],
        out_specs=[pl.BlockSpec((T, 128), lambda t, eid: (t, 0)),
                   pl.BlockSpec((T, 128), lambda t, eid: (t, 0))],
    )
    return pl.pallas_call(
        _expert_body,
        grid_spec=grid_spec,
        out_shape=[jax.ShapeDtypeStruct((npad, 128), F32),
                   jax.ShapeDtypeStruct((npad, 128), F32)],
    )(tile_eid, ysa, ysb,
      We1, be1.reshape(E, 1, WID), We2, be2.reshape(E, 1, WID),
      We3, be3.reshape(E, 1, WID), We4, be4.reshape(E, 1, WID))


def _combine_gather(hsa, hsb, dest3, n):
    per_w = n // NWORK
    nchunks = per_w // CHUNK
    mesh = plsc.VectorSubcoreMesh(core_axis_name="c", subcore_axis_name="s")

    @functools.partial(
        pl.kernel,
        mesh=mesh,
        out_type=[jax.ShapeDtypeStruct((n, 128), F32),
                  jax.ShapeDtypeStruct((n, 128), F32)],
        scratch_types=[
            pltpu.VMEM((nchunks, CHUNK), I32),
            pltpu.VMEM((CHUNK, 128), F32),
            pltpu.VMEM((CHUNK, 128), F32),
            pltpu.SemaphoreType.DMA,
            pltpu.SemaphoreType.DMA,
        ],
    )
    def k(hsa_hbm, hsb_hbm, dest_hbm, outa_hbm, outb_hbm, idx_v, rowa_v,
          rowb_v, sema, semb):
        wid = lax.axis_index("s") * SC_CORES + lax.axis_index("c")
        pltpu.sync_copy(dest_hbm.at[wid], idx_v)
        base = wid * per_w
        for j in range(nchunks):
            ca = pltpu.async_copy(hsa_hbm.at[idx_v.at[j]], rowa_v, sema)
            cb = pltpu.async_copy(hsb_hbm.at[idx_v.at[j]], rowb_v, semb)
            ca.wait()
            cb.wait()
            pltpu.sync_copy(rowa_v, outa_hbm.at[pl.ds(base + j * CHUNK, CHUNK)])
            pltpu.sync_copy(rowb_v, outb_hbm.at[pl.ds(base + j * CHUNK, CHUNK)])

    return k(hsa, hsb, dest3)


def _head_body(hrawa_ref, hrawb_ref, gtop_ref, vdirT_ref,
               wr1az_ref, wvs_ref, wvc_ref, wvi_ref, br1z_ref,
               wr2_ref, br2_ref,
               sig_ref, rgb_ref, ssum_ref):
    i = pl.program_id(0)
    so = jnp.concatenate([hrawa_ref[...], hrawb_ref[...]],
                         axis=1) * gtop_ref[...]
    vt = vdirT_ref[...]
    t12 = jnp.concatenate([vt * (2.0 ** d) for d in range(NDF)], axis=0)
    s = jnp.sin(t12)
    c = jnp.cos(t12)
    u = (jnp.dot(so, wr1az_ref[...], preferred_element_type=F32)
         + lax.dot_general(s, wvs_ref[...], _DN0, preferred_element_type=F32)
         + lax.dot_general(c, wvc_ref[...], _DN0, preferred_element_type=F32)
         + lax.dot_general(vt, wvi_ref[...], _DN0, preferred_element_type=F32)
         + br1z_ref[...])
    z = u[:, 128:129]
    sig = jnp.maximum(z, 0.0) + jnp.log(1.0 + jnp.exp(-jnp.abs(z)))
    sig_ref[...] = sig
    hr = jnp.maximum(u[:, :128], 0.0)
    t = jnp.dot(hr, wr2_ref[...], preferred_element_type=F32) + br2_ref[...]
    rgb_ref[...] = 1.0 / (1.0 + jnp.exp(-t))

    @pl.when(i == 0)
    def _():
        ssum_ref[...] = jnp.zeros_like(ssum_ref)

    ssum_ref[...] += jnp.sum(sig, axis=0, keepdims=True)


def _heads(hrawa, hrawb, gtop, vdirT, wr1az, wvs, wvc, wvi, br1z, wr2p,
           br2p, n):
    bf = 4096
    return pl.pallas_call(
        _head_body,
        grid=(n // bf,),
        in_specs=[
            pl.BlockSpec((bf, 128), lambda i: (i, 0)),
            pl.BlockSpec((bf, 128), lambda i: (i, 0)),
            pl.BlockSpec((bf, 1), lambda i: (i, 0)),
            pl.BlockSpec((3, bf), lambda i: (0, i)),
            pl.BlockSpec((ENC, 256), lambda i: (0, 0)),
            pl.BlockSpec((3 * NDF, 256), lambda i: (0, 0)),
            pl.BlockSpec((3 * NDF, 256), lambda i: (0, 0)),
            pl.BlockSpec((3, 256), lambda i: (0, 0)),
            pl.BlockSpec((1, 256), lambda i: (0, 0)),
            pl.BlockSpec((128, 128), lambda i: (0, 0)),
            pl.BlockSpec((1, 128), lambda i: (0, 0)),
        ],
        out_specs=[
            pl.BlockSpec((bf, 1), lambda i: (i, 0)),
            pl.BlockSpec((bf, 128), lambda i: (i, 0)),
            pl.BlockSpec((1, 1), lambda i: (0, 0)),
        ],
        out_shape=[
            jax.ShapeDtypeStruct((n, 1), F32),
            jax.ShapeDtypeStruct((n, 128), F32),
            jax.ShapeDtypeStruct((1, 1), F32),
        ],
    )(hrawa, hrawb, gtop, vdirT, wr1az, wvs, wvc, wvi,
      br1z, wr2p, br2p)


def kernel(xyz, viewdir, shape_latent, texture_latent, temperature,
           W_enc, b_enc, W_g, b_g,
           We1, be1, We2, be2, We3, be3, We4, be4,
           W_sig, b_sig, W_r1, b_r1, W_r2, b_r2):
    nrays, nsamples, _ = xyz.shape
    n = nrays * nsamples
    npad = (n // T + E) * T

    xyzT = jnp.transpose(xyz, (2, 1, 0)).reshape(3, n)
    vdirT = jnp.transpose(viewdir, (2, 1, 0)).reshape(3, n)
    temp11 = temperature.reshape(1, 1)
    nsf = 3 * NXF
    wsin30 = W_enc[3:3 + nsf]
    wcos30 = W_enc[3 + nsf:3 + 2 * nsf]
    wid3 = W_enc[:3]

    ya, yb, gates, onehot, gtop, counts, gsum = _gating(
        xyzT, temp11, wsin30, wcos30, wid3, b_enc, W_g, b_g, n)

    cnt = counts.reshape(E)
    tile_cnt = jnp.ceil(cnt / T).astype(I32)
    tile_start = jnp.concatenate(
        [jnp.zeros((1,), I32), jnp.cumsum(tile_cnt)[:-1]])
    starts18 = (tile_start * T).astype(F32).reshape(1, E)
    nt = npad // T
    cum = jnp.cumsum(tile_cnt)
    tidx = jnp.arange(nt, dtype=I32)
    tile_eid = jnp.minimum(
        jnp.sum((tidx[:, None] >= cum[None, :]).astype(I32), axis=1),
        E - 1).astype(I32)

    dest = _dest(onehot, starts18, n)
    dest3 = dest.reshape(NWORK, (n // NWORK) // CHUNK, CHUNK)

    ysa, ysb = _dispatch_scatter(ya, yb, dest3, npad)
    hsa, hsb = _experts(tile_eid, ysa, ysb, We1, be1, We2, be2, We3, be3,
                        We4, be4, npad)
    hrawa, hrawb = _combine_gather(hsa, hsb, dest3, n)

    ncf = 3 * NDF
    wr1az = jnp.concatenate(
        [W_r1[:ENC], W_sig, jnp.zeros((ENC, 127), F32)], axis=1)
    wvs = jnp.zeros((ncf, 256), F32).at[:, :128].set(W_r1[ENC + 3:ENC + 3 + ncf])
    wvc = jnp.zeros((ncf, 256), F32).at[:, :128].set(W_r1[ENC + 3 + ncf:])
    wvi = jnp.zeros((3, 256), F32).at[:, :128].set(W_r1[ENC:ENC + 3])
    br1z = jnp.concatenate(
        [b_r1, b_sig, jnp.zeros((127,), F32)]).reshape(1, 256)
    wr2p = jnp.concatenate([W_r2, jnp.zeros((128, 125), F32)], axis=1)
    br2p = jnp.concatenate([b_r2, jnp.zeros((125,), F32)]).reshape(1, 128)

    sig, rgbp, ssum = _heads(hrawa, hrawb, gtop, vdirT, wr1az, wvs, wvc,
                             wvi, br1z, wr2p, br2p, n)

    sigmas = sig.reshape(nsamples, nrays, 1).transpose(1, 0, 2)
    rgbs = rgbp[:, :3].reshape(nsamples, nrays, 3).transpose(1, 0, 2)
    gates_soft_o = gates.reshape(nsamples, nrays, E).transpose(1, 0, 2)
    gates_hard_o = onehot.reshape(nsamples, nrays, E).transpose(1, 0, 2)
    mean_sigma = (ssum / n).reshape(1)
    num_pts = cnt
    aux_loss = E * jnp.sum((cnt / n) * (gsum.reshape(E) / n))
    return (sigmas, rgbs, gates_soft_o, gates_hard_o,
            mean_sigma, num_pts, aux_loss)
